# baseline (speedup 1.0000x reference)
import functools

import jax
import jax.numpy as jnp
from jax import lax
from jax.experimental import pallas as pl
from jax.experimental.pallas import tpu as pltpu
from jax.experimental.pallas import tpu_sc as plsc

EPS = 1e-5
SLOPE = 0.2
K1 = 20
K2 = 40
ROWS = 256
NC = 2
NS = 16


def _stage1_body(xf_ref, xr_ref, Ws_ref, Wl_ref, T_ref, Q_ref, idx_ref):
    b = pl.program_id(0)
    C = xf_ref.shape[1]
    N = xf_ref.shape[2]
    R = xr_ref.shape[2]
    xb = xf_ref[0]
    xr = xr_ref[0]

    dn = (((0,), (1,)), ((), ()))
    A_s = Ws_ref[:, :C]
    A_l = Wl_ref[:, :C]
    B_s = Ws_ref[:, C:] - A_s
    B_l = Wl_ref[:, C:] - A_l
    Ts = lax.dot_general(xr, A_s, dn, preferred_element_type=jnp.float32)
    Tl = lax.dot_general(xr, A_l, dn, preferred_element_type=jnp.float32)
    Qs = lax.dot_general(xr, B_s, dn, preferred_element_type=jnp.float32)
    Ql = lax.dot_general(xr, B_l, dn, preferred_element_type=jnp.float32)
    T_ref[0] = jnp.concatenate([Ts, Tl], axis=1)
    Q_ref[0] = jnp.concatenate([Qs, Ql], axis=1)

    inner = -2.0 * lax.dot_general(xb, xr, (((0,), (0,)), ((), ())),
                                   preferred_element_type=jnp.float32)
    xx_full = jnp.sum(xb * xb, axis=0)
    xx_r = jnp.sum(xr * xr, axis=0)
    pw = -xx_r[None, :] - inner - xx_full[:, None]

    iota = lax.broadcasted_iota(jnp.int32, (N, R), 0)
    off = b * N

    def step(j, vals):
        m = jnp.max(vals, axis=0)
        cand = jnp.where(vals == m[None, :], iota, N)
        i = jnp.min(cand, axis=0)
        idx_ref[0, 0, pl.ds(j, 1), :] = (i + off)[None, :]
        return jnp.where(iota == i[None, :], -jnp.inf, vals)

    lax.fori_loop(0, K2, step, pw)


def _sc_stage2(Tf, idx_flat):
    PTS, D = Tf.shape
    O = D // 2
    NW = NC * NS
    ppw = PTS // NW
    mesh = plsc.VectorSubcoreMesh(core_axis_name="c", subcore_axis_name="s")

    @functools.partial(
        pl.kernel,
        out_type=jax.ShapeDtypeStruct((PTS, 8 * O), jnp.float32),
        mesh=mesh,
        scratch_types=[
            pltpu.VMEM((ppw * K2,), jnp.int32),
            pltpu.VMEM((K2, D), jnp.float32),
            pltpu.VMEM((8 * O,), jnp.float32),
            pltpu.SemaphoreType.DMA,
        ],
    )
    def sc_k(T_hbm, idx_hbm, out_hbm, idx_v, rows_v, orow_v, sem):
        wid = lax.axis_index("s") * NC + lax.axis_index("c")
        base = wid * ppw
        pltpu.sync_copy(idx_hbm.at[pl.ds(base * K2, ppw * K2)], idx_v)

        def point(j, carry):
            pltpu.async_copy(T_hbm.at[idx_v.at[pl.ds(j * K2, K2)]], rows_v,
                             sem).wait()
            for c in range(D // 16):
                s_half = c < (O // 16)
                hi = K1 if s_half else K2
                sl = pl.ds(c * 16, 16)
                v = rows_v[0, sl]
                acc_s = v
                acc_q = v * v
                acc_mx = v
                acc_mn = v
                for r in range(1, hi):
                    v = rows_v[r, sl]
                    acc_s = acc_s + v
                    acc_q = acc_q + v * v
                    acc_mx = jnp.maximum(acc_mx, v)
                    acc_mn = jnp.minimum(acc_mn, v)
                half = 0 if s_half else 4 * O
                cl = c if s_half else c - O // 16
                orow_v[pl.ds(half + cl * 16, 16)] = acc_s
                orow_v[pl.ds(half + O + cl * 16, 16)] = acc_q
                orow_v[pl.ds(half + 2 * O + cl * 16, 16)] = acc_mx
                orow_v[pl.ds(half + 3 * O + cl * 16, 16)] = acc_mn
            pltpu.sync_copy(orow_v, out_hbm.at[base + j])
            return carry

        lax.fori_loop(0, ppw, point, 0)

    return sc_k(Tf, idx_flat)


def _stage3a_body(st_ref, Q_ref, part_ref):
    O = Q_ref.shape[1] // 2
    st = st_ref[...]
    Qs = Q_ref[:, :O]
    Ql = Q_ref[:, O:]
    sum_s, ssq_s = st[:, 0:O], st[:, O:2 * O]
    sum_l, ssq_l = st[:, 4 * O:5 * O], st[:, 5 * O:6 * O]
    tot_s = jnp.sum(sum_s + K1 * Qs, axis=0)
    tot2_s = jnp.sum(ssq_s + 2.0 * Qs * sum_s + K1 * Qs * Qs, axis=0)
    tot_l = jnp.sum(sum_l + K2 * Ql, axis=0)
    tot2_l = jnp.sum(ssq_l + 2.0 * Ql * sum_l + K2 * Ql * Ql, axis=0)
    part_ref[0, 0] = jnp.concatenate([tot_s, tot2_s, tot_l, tot2_l])


def _stage3b_body(st_ref, Q_ref, part_ref, Wf_ref, gs_ref, bs_ref, gl_ref,
                  bl_ref, yf_ref, fpart_ref, *, pts):
    O = Q_ref.shape[1] // 2
    st = st_ref[...]
    Qs = Q_ref[:, :O]
    Ql = Q_ref[:, O:]
    tot = jnp.sum(part_ref[...], axis=0)[0]

    def conv_half(mx_g, mn_g, Q, t, t2, gamma, beta, k):
        cnt = pts * k
        mean = t / cnt
        var = t2 / cnt - mean * mean
        a = gamma * lax.rsqrt(var + EPS)
        c = beta - mean * a
        sel = jnp.where(a >= 0, mx_g, mn_g)
        y = a[None, :] * (sel + Q) + c[None, :]
        return jnp.where(y >= 0, y, SLOPE * y)

    ys = conv_half(st[:, 2 * O:3 * O], st[:, 3 * O:4 * O], Qs,
                   tot[0:O], tot[O:2 * O], gs_ref[0], bs_ref[0], K1)
    yl = conv_half(st[:, 6 * O:7 * O], st[:, 7 * O:8 * O], Ql,
                   tot[2 * O:3 * O], tot[3 * O:4 * O], gl_ref[0], bl_ref[0], K2)
    ycat = jnp.concatenate([ys, yl], axis=1)
    yf = lax.dot_general(ycat, Wf_ref[...], (((1,), (1,)), ((), ())),
                         preferred_element_type=jnp.float32)
    yf_ref[...] = yf
    fpart_ref[0, 0] = jnp.concatenate(
        [jnp.sum(yf, axis=0), jnp.sum(yf * yf, axis=0)])


def _stage3c_body(yf_ref, fpart_ref, gf_ref, bf_ref, out_ref):
    PTS = yf_ref.shape[0]
    yf = yf_ref[...]
    tot = jnp.sum(fpart_ref[...], axis=0)[0]
    O = yf.shape[1]
    m = tot[:O] / PTS
    v = tot[O:] / PTS - m * m
    a = gf_ref[0] * lax.rsqrt(v + EPS)
    c = bf_ref[0] - m * a
    y = a[None, :] * yf + c[None, :]
    out_ref[...] = jnp.where(y >= 0, y, SLOPE * y)


def kernel(x, W_s, gamma_s, beta_s, W_l, gamma_l, beta_l, W_f, gamma_f, beta_f):
    B, C, N = x.shape
    O = W_s.shape[0]
    R = ROWS
    nR = N // R
    PTS = B * N

    T, Qc, idx4 = pl.pallas_call(
        _stage1_body,
        grid=(B, nR),
        in_specs=[
            pl.BlockSpec((1, C, N), lambda b, r: (b, 0, 0)),
            pl.BlockSpec((1, C, R), lambda b, r: (b, 0, r)),
            pl.BlockSpec((O, 2 * C), lambda b, r: (0, 0)),
            pl.BlockSpec((O, 2 * C), lambda b, r: (0, 0)),
        ],
        out_specs=[
            pl.BlockSpec((1, R, 2 * O), lambda b, r: (b, r, 0)),
            pl.BlockSpec((1, R, 2 * O), lambda b, r: (b, r, 0)),
            pl.BlockSpec((1, 1, K2, R), lambda b, r: (b, r, 0, 0)),
        ],
        out_shape=[
            jax.ShapeDtypeStruct((B, N, 2 * O), jnp.float32),
            jax.ShapeDtypeStruct((B, N, 2 * O), jnp.float32),
            jax.ShapeDtypeStruct((B, nR, K2, R), jnp.int32),
        ],
    )(x, x, W_s, W_l)

    idx_flat = idx4.transpose(0, 1, 3, 2).reshape(-1)
    Tf = T.reshape(PTS, 2 * O)
    Qf = Qc.reshape(PTS, 2 * O)

    stats = _sc_stage2(Tf, idx_flat)

    CH = 1024
    nch = PTS // CH
    st_spec = pl.BlockSpec((CH, 8 * O), lambda i: (i, 0))
    q_spec = pl.BlockSpec((CH, 2 * O), lambda i: (i, 0))
    part_spec = pl.BlockSpec((1, 1, 4 * O), lambda i: (i, 0, 0))
    full = lambda shape: pl.BlockSpec(shape, lambda i: tuple(0 for _ in shape))

    part = pl.pallas_call(
        _stage3a_body,
        grid=(nch,),
        in_specs=[st_spec, q_spec],
        out_specs=part_spec,
        out_shape=jax.ShapeDtypeStruct((nch, 1, 4 * O), jnp.float32),
    )(stats, Qf)

    yf, fpart = pl.pallas_call(
        functools.partial(_stage3b_body, pts=PTS),
        grid=(nch,),
        in_specs=[st_spec, q_spec, full((nch, 1, 4 * O)), full((O, 2 * O)),
                  full((1, O)), full((1, O)), full((1, O)), full((1, O))],
        out_specs=[pl.BlockSpec((CH, O), lambda i: (i, 0)),
                   pl.BlockSpec((1, 1, 2 * O), lambda i: (i, 0, 0))],
        out_shape=[jax.ShapeDtypeStruct((PTS, O), jnp.float32),
                   jax.ShapeDtypeStruct((nch, 1, 2 * O), jnp.float32)],
    )(stats, Qf, part, W_f,
      gamma_s.reshape(1, O), beta_s.reshape(1, O),
      gamma_l.reshape(1, O), beta_l.reshape(1, O))

    rows = pl.pallas_call(
        _stage3c_body,
        out_shape=jax.ShapeDtypeStruct((PTS, O), jnp.float32),
    )(yf, fpart, gamma_f.reshape(1, O), beta_f.reshape(1, O))

    return rows.reshape(B, N, O).transpose(0, 2, 1)


# speedup vs baseline: 1.1723x; 1.1723x over previous
import functools

import jax
import jax.numpy as jnp
from jax import lax
from jax.experimental import pallas as pl
from jax.experimental.pallas import tpu as pltpu
from jax.experimental.pallas import tpu_sc as plsc

EPS = 1e-5
SLOPE = 0.2
K1 = 20
K2 = 40
ROWS = 256
NC = 2
NS = 16


def _tree_reduce(a, op):
    s = a.shape[0]
    while s > 1:
        h = s // 2
        a = op(a[:h], a[h:])
        s = h
    return a[0]


def _stage1_body(xf_ref, xr_ref, Ws_ref, Wl_ref, T_ref, Q_ref, idx_ref,
                 vals_ref):
    b = pl.program_id(0)
    C = xf_ref.shape[1]
    N = xf_ref.shape[2]
    R = xr_ref.shape[2]
    xb = xf_ref[0]
    xr = xr_ref[0]

    dn = (((0,), (1,)), ((), ()))
    A_s = Ws_ref[:, :C]
    A_l = Wl_ref[:, :C]
    B_s = Ws_ref[:, C:] - A_s
    B_l = Wl_ref[:, C:] - A_l
    Ts = lax.dot_general(xr, A_s, dn, preferred_element_type=jnp.float32)
    Tl = lax.dot_general(xr, A_l, dn, preferred_element_type=jnp.float32)
    Qs = lax.dot_general(xr, B_s, dn, preferred_element_type=jnp.float32)
    Ql = lax.dot_general(xr, B_l, dn, preferred_element_type=jnp.float32)
    T_ref[0] = jnp.concatenate([Ts, Tl], axis=1)
    Q_ref[0] = jnp.concatenate([Qs, Ql], axis=1)

    inner = -2.0 * lax.dot_general(xb, xr, (((0,), (0,)), ((), ())),
                                   preferred_element_type=jnp.float32)
    xx_full = jnp.sum(xb * xb, axis=0)
    xx_r = jnp.sum(xr * xr, axis=0)
    vals_ref[...] = -xx_r[None, :] - inner - xx_full[:, None]

    iota = lax.broadcasted_iota(jnp.int32, (N, R), 0)
    off = b * N

    def step(j, carry):
        vals = vals_ref[...]
        m = _tree_reduce(vals, jnp.maximum)
        cand = jnp.where(vals == m[None, :], iota, N)
        i = _tree_reduce(cand, jnp.minimum)
        idx_ref[0, 0, pl.ds(j, 1), :] = (i + off)[None, :]
        vals_ref[...] = jnp.where(iota == i[None, :], -jnp.inf, vals)
        return carry

    lax.fori_loop(0, K2, step, 0)


def _sc_stage2(Tf, idx_flat):
    PTS, D = Tf.shape
    O = D // 2
    NW = NC * NS
    ppw = PTS // NW
    mesh = plsc.VectorSubcoreMesh(core_axis_name="c", subcore_axis_name="s")

    @functools.partial(
        pl.kernel,
        out_type=jax.ShapeDtypeStruct((PTS, 8 * O), jnp.float32),
        mesh=mesh,
        scratch_types=[
            pltpu.VMEM((ppw * K2,), jnp.int32),
            pltpu.VMEM((K2, D), jnp.float32),
            pltpu.VMEM((8 * O,), jnp.float32),
            pltpu.SemaphoreType.DMA,
        ],
    )
    def sc_k(T_hbm, idx_hbm, out_hbm, idx_v, rows_v, orow_v, sem):
        wid = lax.axis_index("s") * NC + lax.axis_index("c")
        base = wid * ppw
        pltpu.sync_copy(idx_hbm.at[pl.ds(base * K2, ppw * K2)], idx_v)

        def point(j, carry):
            pltpu.async_copy(T_hbm.at[idx_v.at[pl.ds(j * K2, K2)]], rows_v,
                             sem).wait()
            for c in range(D // 16):
                s_half = c < (O // 16)
                hi = K1 if s_half else K2
                sl = pl.ds(c * 16, 16)
                v = rows_v[0, sl]
                acc_s = v
                acc_q = v * v
                acc_mx = v
                acc_mn = v
                for r in range(1, hi):
                    v = rows_v[r, sl]
                    acc_s = acc_s + v
                    acc_q = acc_q + v * v
                    acc_mx = jnp.maximum(acc_mx, v)
                    acc_mn = jnp.minimum(acc_mn, v)
                half = 0 if s_half else 4 * O
                cl = c if s_half else c - O // 16
                orow_v[pl.ds(half + cl * 16, 16)] = acc_s
                orow_v[pl.ds(half + O + cl * 16, 16)] = acc_q
                orow_v[pl.ds(half + 2 * O + cl * 16, 16)] = acc_mx
                orow_v[pl.ds(half + 3 * O + cl * 16, 16)] = acc_mn
            pltpu.sync_copy(orow_v, out_hbm.at[base + j])
            return carry

        lax.fori_loop(0, ppw, point, 0)

    return sc_k(Tf, idx_flat)


def _stage3a_body(st_ref, Q_ref, part_ref):
    O = Q_ref.shape[1] // 2
    st = st_ref[...]
    Qs = Q_ref[:, :O]
    Ql = Q_ref[:, O:]
    sum_s, ssq_s = st[:, 0:O], st[:, O:2 * O]
    sum_l, ssq_l = st[:, 4 * O:5 * O], st[:, 5 * O:6 * O]
    tot_s = jnp.sum(sum_s + K1 * Qs, axis=0)
    tot2_s = jnp.sum(ssq_s + 2.0 * Qs * sum_s + K1 * Qs * Qs, axis=0)
    tot_l = jnp.sum(sum_l + K2 * Ql, axis=0)
    tot2_l = jnp.sum(ssq_l + 2.0 * Ql * sum_l + K2 * Ql * Ql, axis=0)
    part_ref[0, 0] = jnp.concatenate([tot_s, tot2_s, tot_l, tot2_l])


def _stage3b_body(st_ref, Q_ref, part_ref, Wf_ref, gs_ref, bs_ref, gl_ref,
                  bl_ref, yf_ref, fpart_ref, *, pts):
    O = Q_ref.shape[1] // 2
    st = st_ref[...]
    Qs = Q_ref[:, :O]
    Ql = Q_ref[:, O:]
    tot = jnp.sum(part_ref[...], axis=0)[0]

    def conv_half(mx_g, mn_g, Q, t, t2, gamma, beta, k):
        cnt = pts * k
        mean = t / cnt
        var = t2 / cnt - mean * mean
        a = gamma * lax.rsqrt(var + EPS)
        c = beta - mean * a
        sel = jnp.where(a >= 0, mx_g, mn_g)
        y = a[None, :] * (sel + Q) + c[None, :]
        return jnp.where(y >= 0, y, SLOPE * y)

    ys = conv_half(st[:, 2 * O:3 * O], st[:, 3 * O:4 * O], Qs,
                   tot[0:O], tot[O:2 * O], gs_ref[0], bs_ref[0], K1)
    yl = conv_half(st[:, 6 * O:7 * O], st[:, 7 * O:8 * O], Ql,
                   tot[2 * O:3 * O], tot[3 * O:4 * O], gl_ref[0], bl_ref[0], K2)
    ycat = jnp.concatenate([ys, yl], axis=1)
    yf = lax.dot_general(ycat, Wf_ref[...], (((1,), (1,)), ((), ())),
                         preferred_element_type=jnp.float32)
    yf_ref[...] = yf
    fpart_ref[0, 0] = jnp.concatenate(
        [jnp.sum(yf, axis=0), jnp.sum(yf * yf, axis=0)])


def _stage3c_body(yf_ref, fpart_ref, gf_ref, bf_ref, out_ref):
    PTS = yf_ref.shape[0]
    yf = yf_ref[...]
    tot = jnp.sum(fpart_ref[...], axis=0)[0]
    O = yf.shape[1]
    m = tot[:O] / PTS
    v = tot[O:] / PTS - m * m
    a = gf_ref[0] * lax.rsqrt(v + EPS)
    c = bf_ref[0] - m * a
    y = a[None, :] * yf + c[None, :]
    out_ref[...] = jnp.where(y >= 0, y, SLOPE * y)


def kernel(x, W_s, gamma_s, beta_s, W_l, gamma_l, beta_l, W_f, gamma_f, beta_f):
    B, C, N = x.shape
    O = W_s.shape[0]
    R = ROWS
    nR = N // R
    PTS = B * N

    T, Qc, idx4 = pl.pallas_call(
        _stage1_body,
        grid=(B, nR),
        in_specs=[
            pl.BlockSpec((1, C, N), lambda b, r: (b, 0, 0)),
            pl.BlockSpec((1, C, R), lambda b, r: (b, 0, r)),
            pl.BlockSpec((O, 2 * C), lambda b, r: (0, 0)),
            pl.BlockSpec((O, 2 * C), lambda b, r: (0, 0)),
        ],
        out_specs=[
            pl.BlockSpec((1, R, 2 * O), lambda b, r: (b, r, 0)),
            pl.BlockSpec((1, R, 2 * O), lambda b, r: (b, r, 0)),
            pl.BlockSpec((1, 1, K2, R), lambda b, r: (b, r, 0, 0)),
        ],
        out_shape=[
            jax.ShapeDtypeStruct((B, N, 2 * O), jnp.float32),
            jax.ShapeDtypeStruct((B, N, 2 * O), jnp.float32),
            jax.ShapeDtypeStruct((B, nR, K2, R), jnp.int32),
        ],
        scratch_shapes=[pltpu.VMEM((N, R), jnp.float32)],
    )(x, x, W_s, W_l)

    idx_flat = idx4.transpose(0, 1, 3, 2).reshape(-1)
    Tf = T.reshape(PTS, 2 * O)
    Qf = Qc.reshape(PTS, 2 * O)

    stats = _sc_stage2(Tf, idx_flat)

    CH = 1024
    nch = PTS // CH
    st_spec = pl.BlockSpec((CH, 8 * O), lambda i: (i, 0))
    q_spec = pl.BlockSpec((CH, 2 * O), lambda i: (i, 0))
    part_spec = pl.BlockSpec((1, 1, 4 * O), lambda i: (i, 0, 0))
    full = lambda shape: pl.BlockSpec(shape, lambda i: tuple(0 for _ in shape))

    part = pl.pallas_call(
        _stage3a_body,
        grid=(nch,),
        in_specs=[st_spec, q_spec],
        out_specs=part_spec,
        out_shape=jax.ShapeDtypeStruct((nch, 1, 4 * O), jnp.float32),
    )(stats, Qf)

    yf, fpart = pl.pallas_call(
        functools.partial(_stage3b_body, pts=PTS),
        grid=(nch,),
        in_specs=[st_spec, q_spec, full((nch, 1, 4 * O)), full((O, 2 * O)),
                  full((1, O)), full((1, O)), full((1, O)), full((1, O))],
        out_specs=[pl.BlockSpec((CH, O), lambda i: (i, 0)),
                   pl.BlockSpec((1, 1, 2 * O), lambda i: (i, 0, 0))],
        out_shape=[jax.ShapeDtypeStruct((PTS, O), jnp.float32),
                   jax.ShapeDtypeStruct((nch, 1, 2 * O), jnp.float32)],
    )(stats, Qf, part, W_f,
      gamma_s.reshape(1, O), beta_s.reshape(1, O),
      gamma_l.reshape(1, O), beta_l.reshape(1, O))

    rows = pl.pallas_call(
        _stage3c_body,
        out_shape=jax.ShapeDtypeStruct((PTS, O), jnp.float32),
    )(yf, fpart, gamma_f.reshape(1, O), beta_f.reshape(1, O))

    return rows.reshape(B, N, O).transpose(0, 2, 1)


# speedup vs baseline: 1.2961x; 1.1055x over previous
import functools

import jax
import jax.numpy as jnp
from jax import lax
from jax.experimental import pallas as pl
from jax.experimental.pallas import tpu as pltpu
from jax.experimental.pallas import tpu_sc as plsc

EPS = 1e-5
SLOPE = 0.2
K1 = 20
K2 = 40
ROWS = 256
NC = 2
NS = 16


def _tree_reduce(a, op):
    s = a.shape[0]
    while s > 1:
        h = s // 2
        a = op(a[:h], a[h:])
        s = h
    return a[0]


def _stage1_body(xf_ref, xr_ref, Ws_ref, Wl_ref, T_ref, Q_ref, idx_ref,
                 vals_ref):
    b = pl.program_id(0)
    C = xf_ref.shape[1]
    N = xf_ref.shape[2]
    R = xr_ref.shape[2]
    xb = xf_ref[0]
    xr = xr_ref[0]

    dn = (((0,), (1,)), ((), ()))
    A_s = Ws_ref[:, :C]
    A_l = Wl_ref[:, :C]
    B_s = Ws_ref[:, C:] - A_s
    B_l = Wl_ref[:, C:] - A_l
    Ts = lax.dot_general(xr, A_s, dn, preferred_element_type=jnp.float32)
    Tl = lax.dot_general(xr, A_l, dn, preferred_element_type=jnp.float32)
    Qs = lax.dot_general(xr, B_s, dn, preferred_element_type=jnp.float32)
    Ql = lax.dot_general(xr, B_l, dn, preferred_element_type=jnp.float32)
    T_ref[0] = jnp.concatenate([Ts, Tl], axis=1)
    Q_ref[0] = jnp.concatenate([Qs, Ql], axis=1)

    inner = -2.0 * lax.dot_general(xb, xr, (((0,), (0,)), ((), ())),
                                   preferred_element_type=jnp.float32)
    xx_full = jnp.sum(xb * xb, axis=0)
    xx_r = jnp.sum(xr * xr, axis=0)
    vals_ref[...] = -xx_r[None, :] - inner - xx_full[:, None]

    iota = lax.broadcasted_iota(jnp.int32, (N, R), 0)
    off = b * N

    def step(j, i_prev):
        vals = jnp.where(iota == i_prev[None, :], -jnp.inf, vals_ref[...])
        vals_ref[...] = vals
        m = _tree_reduce(vals, jnp.maximum)
        cand = jnp.where(vals == m[None, :], iota, N)
        i = _tree_reduce(cand, jnp.minimum)
        idx_ref[0, 0, pl.ds(j, 1), :] = (i + off)[None, :]
        return i

    lax.fori_loop(0, K2, step, jnp.full((R,), N, jnp.int32))


def _sc_stage2(Tf, idx_flat):
    PTS, D = Tf.shape
    O = D // 2
    NW = NC * NS
    ppw = PTS // NW
    mesh = plsc.VectorSubcoreMesh(core_axis_name="c", subcore_axis_name="s")

    @functools.partial(
        pl.kernel,
        out_type=jax.ShapeDtypeStruct((PTS, 8 * O), jnp.float32),
        mesh=mesh,
        scratch_types=[
            pltpu.VMEM((ppw * K2,), jnp.int32),
            pltpu.VMEM((K2, D), jnp.float32),
            pltpu.VMEM((8 * O,), jnp.float32),
            pltpu.SemaphoreType.DMA,
        ],
    )
    def sc_k(T_hbm, idx_hbm, out_hbm, idx_v, rows_v, orow_v, sem):
        wid = lax.axis_index("s") * NC + lax.axis_index("c")
        base = wid * ppw
        pltpu.sync_copy(idx_hbm.at[pl.ds(base * K2, ppw * K2)], idx_v)

        def point(j, carry):
            pltpu.async_copy(T_hbm.at[idx_v.at[pl.ds(j * K2, K2)]], rows_v,
                             sem).wait()
            for c in range(D // 16):
                s_half = c < (O // 16)
                hi = K1 if s_half else K2
                sl = pl.ds(c * 16, 16)
                v = rows_v[0, sl]
                acc_s = v
                acc_q = v * v
                acc_mx = v
                acc_mn = v
                for r in range(1, hi):
                    v = rows_v[r, sl]
                    acc_s = acc_s + v
                    acc_q = acc_q + v * v
                    acc_mx = jnp.maximum(acc_mx, v)
                    acc_mn = jnp.minimum(acc_mn, v)
                half = 0 if s_half else 4 * O
                cl = c if s_half else c - O // 16
                orow_v[pl.ds(half + cl * 16, 16)] = acc_s
                orow_v[pl.ds(half + O + cl * 16, 16)] = acc_q
                orow_v[pl.ds(half + 2 * O + cl * 16, 16)] = acc_mx
                orow_v[pl.ds(half + 3 * O + cl * 16, 16)] = acc_mn
            pltpu.sync_copy(orow_v, out_hbm.at[base + j])
            return carry

        lax.fori_loop(0, ppw, point, 0)

    return sc_k(Tf, idx_flat)


def _stage3a_body(st_ref, Q_ref, part_ref):
    O = Q_ref.shape[1] // 2
    st = st_ref[...]
    Qs = Q_ref[:, :O]
    Ql = Q_ref[:, O:]
    sum_s, ssq_s = st[:, 0:O], st[:, O:2 * O]
    sum_l, ssq_l = st[:, 4 * O:5 * O], st[:, 5 * O:6 * O]
    tot_s = jnp.sum(sum_s + K1 * Qs, axis=0)
    tot2_s = jnp.sum(ssq_s + 2.0 * Qs * sum_s + K1 * Qs * Qs, axis=0)
    tot_l = jnp.sum(sum_l + K2 * Ql, axis=0)
    tot2_l = jnp.sum(ssq_l + 2.0 * Ql * sum_l + K2 * Ql * Ql, axis=0)
    part_ref[0, 0] = jnp.concatenate([tot_s, tot2_s, tot_l, tot2_l])


def _stage3b_body(st_ref, Q_ref, part_ref, Wf_ref, gs_ref, bs_ref, gl_ref,
                  bl_ref, yf_ref, fpart_ref, *, pts):
    O = Q_ref.shape[1] // 2
    st = st_ref[...]
    Qs = Q_ref[:, :O]
    Ql = Q_ref[:, O:]
    tot = jnp.sum(part_ref[...], axis=0)[0]

    def conv_half(mx_g, mn_g, Q, t, t2, gamma, beta, k):
        cnt = pts * k
        mean = t / cnt
        var = t2 / cnt - mean * mean
        a = gamma * lax.rsqrt(var + EPS)
        c = beta - mean * a
        sel = jnp.where(a >= 0, mx_g, mn_g)
        y = a[None, :] * (sel + Q) + c[None, :]
        return jnp.where(y >= 0, y, SLOPE * y)

    ys = conv_half(st[:, 2 * O:3 * O], st[:, 3 * O:4 * O], Qs,
                   tot[0:O], tot[O:2 * O], gs_ref[0], bs_ref[0], K1)
    yl = conv_half(st[:, 6 * O:7 * O], st[:, 7 * O:8 * O], Ql,
                   tot[2 * O:3 * O], tot[3 * O:4 * O], gl_ref[0], bl_ref[0], K2)
    ycat = jnp.concatenate([ys, yl], axis=1)
    yf = lax.dot_general(ycat, Wf_ref[...], (((1,), (1,)), ((), ())),
                         preferred_element_type=jnp.float32)
    yf_ref[...] = yf
    fpart_ref[0, 0] = jnp.concatenate(
        [jnp.sum(yf, axis=0), jnp.sum(yf * yf, axis=0)])


def _stage3c_body(yf_ref, fpart_ref, gf_ref, bf_ref, out_ref):
    PTS = yf_ref.shape[0]
    yf = yf_ref[...]
    tot = jnp.sum(fpart_ref[...], axis=0)[0]
    O = yf.shape[1]
    m = tot[:O] / PTS
    v = tot[O:] / PTS - m * m
    a = gf_ref[0] * lax.rsqrt(v + EPS)
    c = bf_ref[0] - m * a
    y = a[None, :] * yf + c[None, :]
    out_ref[...] = jnp.where(y >= 0, y, SLOPE * y)


def kernel(x, W_s, gamma_s, beta_s, W_l, gamma_l, beta_l, W_f, gamma_f, beta_f):
    B, C, N = x.shape
    O = W_s.shape[0]
    R = ROWS
    nR = N // R
    PTS = B * N

    T, Qc, idx4 = pl.pallas_call(
        _stage1_body,
        grid=(B, nR),
        in_specs=[
            pl.BlockSpec((1, C, N), lambda b, r: (b, 0, 0)),
            pl.BlockSpec((1, C, R), lambda b, r: (b, 0, r)),
            pl.BlockSpec((O, 2 * C), lambda b, r: (0, 0)),
            pl.BlockSpec((O, 2 * C), lambda b, r: (0, 0)),
        ],
        out_specs=[
            pl.BlockSpec((1, R, 2 * O), lambda b, r: (b, r, 0)),
            pl.BlockSpec((1, R, 2 * O), lambda b, r: (b, r, 0)),
            pl.BlockSpec((1, 1, K2, R), lambda b, r: (b, r, 0, 0)),
        ],
        out_shape=[
            jax.ShapeDtypeStruct((B, N, 2 * O), jnp.float32),
            jax.ShapeDtypeStruct((B, N, 2 * O), jnp.float32),
            jax.ShapeDtypeStruct((B, nR, K2, R), jnp.int32),
        ],
        scratch_shapes=[pltpu.VMEM((N, R), jnp.float32)],
    )(x, x, W_s, W_l)

    idx_flat = idx4.transpose(0, 1, 3, 2).reshape(-1)
    Tf = T.reshape(PTS, 2 * O)
    Qf = Qc.reshape(PTS, 2 * O)

    stats = _sc_stage2(Tf, idx_flat)

    CH = 1024
    nch = PTS // CH
    st_spec = pl.BlockSpec((CH, 8 * O), lambda i: (i, 0))
    q_spec = pl.BlockSpec((CH, 2 * O), lambda i: (i, 0))
    part_spec = pl.BlockSpec((1, 1, 4 * O), lambda i: (i, 0, 0))
    full = lambda shape: pl.BlockSpec(shape, lambda i: tuple(0 for _ in shape))

    part = pl.pallas_call(
        _stage3a_body,
        grid=(nch,),
        in_specs=[st_spec, q_spec],
        out_specs=part_spec,
        out_shape=jax.ShapeDtypeStruct((nch, 1, 4 * O), jnp.float32),
    )(stats, Qf)

    yf, fpart = pl.pallas_call(
        functools.partial(_stage3b_body, pts=PTS),
        grid=(nch,),
        in_specs=[st_spec, q_spec, full((nch, 1, 4 * O)), full((O, 2 * O)),
                  full((1, O)), full((1, O)), full((1, O)), full((1, O))],
        out_specs=[pl.BlockSpec((CH, O), lambda i: (i, 0)),
                   pl.BlockSpec((1, 1, 2 * O), lambda i: (i, 0, 0))],
        out_shape=[jax.ShapeDtypeStruct((PTS, O), jnp.float32),
                   jax.ShapeDtypeStruct((nch, 1, 2 * O), jnp.float32)],
    )(stats, Qf, part, W_f,
      gamma_s.reshape(1, O), beta_s.reshape(1, O),
      gamma_l.reshape(1, O), beta_l.reshape(1, O))

    rows = pl.pallas_call(
        _stage3c_body,
        out_shape=jax.ShapeDtypeStruct((PTS, O), jnp.float32),
    )(yf, fpart, gamma_f.reshape(1, O), beta_f.reshape(1, O))

    return rows.reshape(B, N, O).transpose(0, 2, 1)


# speedup vs baseline: 2.1927x; 1.6918x over previous
import functools

import jax
import jax.numpy as jnp
from jax import lax
from jax.experimental import pallas as pl
from jax.experimental.pallas import tpu as pltpu
from jax.experimental.pallas import tpu_sc as plsc

EPS = 1e-5
SLOPE = 0.2
K1 = 20
K2 = 40
ROWS = 256
NC = 2
NS = 16


def _tree_reduce(a, op):
    s = a.shape[0]
    while s > 1:
        h = s // 2
        a = op(a[:h], a[h:])
        s = h
    return a[0]


def _stage1_body(xf_ref, xr_ref, Ws_ref, Wl_ref, T_ref, Q_ref,
                 idx_ref, vals_ref):
    b = pl.program_id(0)
    C = xf_ref.shape[1]
    N = xf_ref.shape[2]
    R = xr_ref.shape[2]
    xb = xf_ref[0]
    xr = xr_ref[0]

    dn = (((0,), (1,)), ((), ()))
    A_s = Ws_ref[:, :C]
    A_l = Wl_ref[:, :C]
    B_s = Ws_ref[:, C:] - A_s
    B_l = Wl_ref[:, C:] - A_l
    Ts = lax.dot_general(xr, A_s, dn, preferred_element_type=jnp.float32)
    Tl = lax.dot_general(xr, A_l, dn, preferred_element_type=jnp.float32)
    Qs = lax.dot_general(xr, B_s, dn, preferred_element_type=jnp.float32)
    Ql = lax.dot_general(xr, B_l, dn, preferred_element_type=jnp.float32)
    T_ref[0] = jnp.concatenate([Ts, Tl], axis=1)
    Q_ref[0] = jnp.concatenate([Qs, Ql], axis=1)

    inner = -2.0 * lax.dot_general(xb, xr, (((0,), (0,)), ((), ())),
                                   preferred_element_type=jnp.float32)
    xx_full = jnp.sum(xb * xb, axis=0)
    xx_r = jnp.sum(xr * xr, axis=0)
    vals_ref[...] = -xx_r[None, :] - inner - xx_full[:, None]

    iota = lax.broadcasted_iota(jnp.int32, (N, R), 0)
    off = b * N

    def step(j, i_prev):
        vals = jnp.where(iota == i_prev[None, :], -jnp.inf, vals_ref[...])
        vals_ref[...] = vals
        v, ii = vals, iota
        s = N
        while s > 1:
            h = s // 2
            take_hi = v[h:] > v[:h]
            v = jnp.where(take_hi, v[h:], v[:h])
            ii = jnp.where(take_hi, ii[h:], ii[:h])
            s = h
        i = ii[0]
        idx_ref[0, 0, pl.ds(j, 1), :] = (i + off)[None, :]
        return i

    lax.fori_loop(0, K2, step, jnp.full((R,), N, jnp.int32))


def _sc_stage2(Tf, idx_flat):
    PTS, D = Tf.shape
    O = D // 2
    NW = NC * NS
    ppw = PTS // NW
    ngrp = ppw // 2
    mesh = plsc.VectorSubcoreMesh(core_axis_name="c", subcore_axis_name="s")

    @functools.partial(
        pl.kernel,
        out_type=jax.ShapeDtypeStruct((PTS, 8 * O), jnp.float32),
        mesh=mesh,
        scratch_types=[
            pltpu.VMEM((ppw * K2,), jnp.int32),
            pltpu.VMEM((K2, D), jnp.float32),
            pltpu.VMEM((K2, D), jnp.float32),
            pltpu.VMEM((8 * O,), jnp.float32),
            pltpu.SemaphoreType.DMA,
            pltpu.SemaphoreType.DMA,
        ],
    )
    def sc_k(T_hbm, idx_hbm, out_hbm, idx_v, rv0, rv1, orow_v, sem0, sem1):
        wid = lax.axis_index("s") * NC + lax.axis_index("c")
        base = wid * ppw
        pltpu.sync_copy(idx_hbm.at[pl.ds(base * K2, ppw * K2)], idx_v)

        def issue(p, rv, sem):
            pltpu.async_copy(T_hbm.at[idx_v.at[pl.ds(p * K2, K2)]], rv, sem)

        def drain(rv, sem):
            pltpu.make_async_copy(T_hbm.at[pl.ds(0, K2)], rv, sem).wait()

        def compute(p, rv):
            for c in range(D // 16):
                s_half = c < (O // 16)
                hi = K1 if s_half else K2
                sl = pl.ds(c * 16, 16)
                v = rv[0, sl]
                acc_s = v
                acc_q = v * v
                acc_mx = v
                acc_mn = v
                for r in range(1, hi):
                    v = rv[r, sl]
                    acc_s = acc_s + v
                    acc_q = acc_q + v * v
                    acc_mx = jnp.maximum(acc_mx, v)
                    acc_mn = jnp.minimum(acc_mn, v)
                half = 0 if s_half else 4 * O
                cl = c if s_half else c - O // 16
                orow_v[pl.ds(half + cl * 16, 16)] = acc_s
                orow_v[pl.ds(half + O + cl * 16, 16)] = acc_q
                orow_v[pl.ds(half + 2 * O + cl * 16, 16)] = acc_mx
                orow_v[pl.ds(half + 3 * O + cl * 16, 16)] = acc_mn
            pltpu.sync_copy(orow_v, out_hbm.at[base + p])

        issue(0, rv0, sem0)

        def group(g, carry):
            p0 = 2 * g
            issue(p0 + 1, rv1, sem1)
            drain(rv0, sem0)
            compute(p0, rv0)

            @pl.when(g + 1 < ngrp)
            def _():
                issue(p0 + 2, rv0, sem0)

            drain(rv1, sem1)
            compute(p0 + 1, rv1)
            return carry

        lax.fori_loop(0, ngrp, group, 0)

    return sc_k(Tf, idx_flat)


def _stage3a_body(st_ref, Q_ref, part_ref):
    O = Q_ref.shape[1] // 2
    st = st_ref[...]
    Qs = Q_ref[:, :O]
    Ql = Q_ref[:, O:]
    sum_s, ssq_s = st[:, 0:O], st[:, O:2 * O]
    sum_l, ssq_l = st[:, 4 * O:5 * O], st[:, 5 * O:6 * O]
    tot_s = jnp.sum(sum_s + K1 * Qs, axis=0)
    tot2_s = jnp.sum(ssq_s + 2.0 * Qs * sum_s + K1 * Qs * Qs, axis=0)
    tot_l = jnp.sum(sum_l + K2 * Ql, axis=0)
    tot2_l = jnp.sum(ssq_l + 2.0 * Ql * sum_l + K2 * Ql * Ql, axis=0)
    part_ref[0, 0] = jnp.concatenate([tot_s, tot2_s, tot_l, tot2_l])


def _stage3b_body(st_ref, Q_ref, part_ref, Wf_ref, gs_ref, bs_ref, gl_ref,
                  bl_ref, yf_ref, fpart_ref, *, pts):
    O = Q_ref.shape[1] // 2
    st = st_ref[...]
    Qs = Q_ref[:, :O]
    Ql = Q_ref[:, O:]
    tot = jnp.sum(part_ref[...], axis=0)[0]

    def conv_half(mx_g, mn_g, Q, t, t2, gamma, beta, k):
        cnt = pts * k
        mean = t / cnt
        var = t2 / cnt - mean * mean
        a = gamma * lax.rsqrt(var + EPS)
        c = beta - mean * a
        sel = jnp.where(a >= 0, mx_g, mn_g)
        y = a[None, :] * (sel + Q) + c[None, :]
        return jnp.where(y >= 0, y, SLOPE * y)

    ys = conv_half(st[:, 2 * O:3 * O], st[:, 3 * O:4 * O], Qs,
                   tot[0:O], tot[O:2 * O], gs_ref[0], bs_ref[0], K1)
    yl = conv_half(st[:, 6 * O:7 * O], st[:, 7 * O:8 * O], Ql,
                   tot[2 * O:3 * O], tot[3 * O:4 * O], gl_ref[0], bl_ref[0], K2)
    ycat = jnp.concatenate([ys, yl], axis=1)
    yf = lax.dot_general(ycat, Wf_ref[...], (((1,), (1,)), ((), ())),
                         preferred_element_type=jnp.float32)
    yf_ref[...] = yf
    fpart_ref[0, 0] = jnp.concatenate(
        [jnp.sum(yf, axis=0), jnp.sum(yf * yf, axis=0)])


def _stage3c_body(yf_ref, fpart_ref, gf_ref, bf_ref, out_ref):
    PTS = yf_ref.shape[0]
    yf = yf_ref[...]
    tot = jnp.sum(fpart_ref[...], axis=0)[0]
    O = yf.shape[1]
    m = tot[:O] / PTS
    v = tot[O:] / PTS - m * m
    a = gf_ref[0] * lax.rsqrt(v + EPS)
    c = bf_ref[0] - m * a
    y = a[None, :] * yf + c[None, :]
    out_ref[...] = jnp.where(y >= 0, y, SLOPE * y)


def kernel(x, W_s, gamma_s, beta_s, W_l, gamma_l, beta_l, W_f, gamma_f, beta_f):
    B, C, N = x.shape
    O = W_s.shape[0]
    R = ROWS
    nR = N // R
    PTS = B * N

    T, Qc, idx4 = pl.pallas_call(
        _stage1_body,
        grid=(B, nR),
        in_specs=[
            pl.BlockSpec((1, C, N), lambda b, r: (b, 0, 0)),
            pl.BlockSpec((1, C, R), lambda b, r: (b, 0, r)),
            pl.BlockSpec((O, 2 * C), lambda b, r: (0, 0)),
            pl.BlockSpec((O, 2 * C), lambda b, r: (0, 0)),
        ],
        out_specs=[
            pl.BlockSpec((1, R, 2 * O), lambda b, r: (b, r, 0)),
            pl.BlockSpec((1, R, 2 * O), lambda b, r: (b, r, 0)),
            pl.BlockSpec((1, 1, K2, R), lambda b, r: (b, r, 0, 0)),
        ],
        out_shape=[
            jax.ShapeDtypeStruct((B, N, 2 * O), jnp.float32),
            jax.ShapeDtypeStruct((B, N, 2 * O), jnp.float32),
            jax.ShapeDtypeStruct((B, nR, K2, R), jnp.int32),
        ],
        scratch_shapes=[pltpu.VMEM((N, R), jnp.float32)],
    )(x, x, W_s, W_l)

    idx_flat = idx4.transpose(0, 1, 3, 2).reshape(-1)
    Qf = Qc.reshape(PTS, 2 * O)

    stats = _sc_stage2(T.reshape(PTS, 2 * O), idx_flat)

    CH = 1024
    nch = PTS // CH
    st_spec = pl.BlockSpec((CH, 8 * O), lambda i: (i, 0))
    q_spec = pl.BlockSpec((CH, 2 * O), lambda i: (i, 0))
    part_spec = pl.BlockSpec((1, 1, 4 * O), lambda i: (i, 0, 0))
    full = lambda shape: pl.BlockSpec(shape, lambda i: tuple(0 for _ in shape))

    part = pl.pallas_call(
        _stage3a_body,
        grid=(nch,),
        in_specs=[st_spec, q_spec],
        out_specs=part_spec,
        out_shape=jax.ShapeDtypeStruct((nch, 1, 4 * O), jnp.float32),
    )(stats, Qf)

    yf, fpart = pl.pallas_call(
        functools.partial(_stage3b_body, pts=PTS),
        grid=(nch,),
        in_specs=[st_spec, q_spec, full((nch, 1, 4 * O)), full((O, 2 * O)),
                  full((1, O)), full((1, O)), full((1, O)), full((1, O))],
        out_specs=[pl.BlockSpec((CH, O), lambda i: (i, 0)),
                   pl.BlockSpec((1, 1, 2 * O), lambda i: (i, 0, 0))],
        out_shape=[jax.ShapeDtypeStruct((PTS, O), jnp.float32),
                   jax.ShapeDtypeStruct((nch, 1, 2 * O), jnp.float32)],
    )(stats, Qf, part, W_f,
      gamma_s.reshape(1, O), beta_s.reshape(1, O),
      gamma_l.reshape(1, O), beta_l.reshape(1, O))

    rows = pl.pallas_call(
        _stage3c_body,
        out_shape=jax.ShapeDtypeStruct((PTS, O), jnp.float32),
    )(yf, fpart, gamma_f.reshape(1, O), beta_f.reshape(1, O))

    return rows.reshape(B, N, O).transpose(0, 2, 1)


# speedup vs baseline: 2.2996x; 1.0488x over previous
import functools

import jax
import jax.numpy as jnp
from jax import lax
from jax.experimental import pallas as pl
from jax.experimental.pallas import tpu as pltpu
from jax.experimental.pallas import tpu_sc as plsc

EPS = 1e-5
SLOPE = 0.2
K1 = 20
K2 = 40
ROWS = 512
NC = 2
NS = 16


def _tree_reduce(a, op):
    s = a.shape[0]
    while s > 1:
        h = s // 2
        a = op(a[:h], a[h:])
        s = h
    return a[0]


def _stage1_body(xf_ref, xr_ref, Ws_ref, Wl_ref, T_ref, Q_ref,
                 idx_ref, vals_ref):
    b = pl.program_id(0)
    C = xf_ref.shape[1]
    N = xf_ref.shape[2]
    R = xr_ref.shape[2]
    xb = xf_ref[0]
    xr = xr_ref[0]

    dn = (((0,), (1,)), ((), ()))
    A_s = Ws_ref[:, :C]
    A_l = Wl_ref[:, :C]
    B_s = Ws_ref[:, C:] - A_s
    B_l = Wl_ref[:, C:] - A_l
    Ts = lax.dot_general(xr, A_s, dn, preferred_element_type=jnp.float32)
    Tl = lax.dot_general(xr, A_l, dn, preferred_element_type=jnp.float32)
    Qs = lax.dot_general(xr, B_s, dn, preferred_element_type=jnp.float32)
    Ql = lax.dot_general(xr, B_l, dn, preferred_element_type=jnp.float32)
    T_ref[0] = jnp.concatenate([Ts, Tl], axis=1)
    Q_ref[0] = jnp.concatenate([Qs, Ql], axis=1)

    inner = -2.0 * lax.dot_general(xb, xr, (((0,), (0,)), ((), ())),
                                   preferred_element_type=jnp.float32)
    xx_full = jnp.sum(xb * xb, axis=0)
    xx_r = jnp.sum(xr * xr, axis=0)
    vals_ref[...] = -xx_r[None, :] - inner - xx_full[:, None]

    iota = lax.broadcasted_iota(jnp.int32, (N, R), 0)
    off = b * N

    def step(j, i_prev):
        vals = jnp.where(iota == i_prev[None, :], -jnp.inf, vals_ref[...])
        vals_ref[...] = vals
        v, ii = vals, iota
        s = N
        while s > 1:
            h = s // 2
            take_hi = v[h:] > v[:h]
            v = jnp.where(take_hi, v[h:], v[:h])
            ii = jnp.where(take_hi, ii[h:], ii[:h])
            s = h
        i = ii[0]
        idx_ref[0, 0, pl.ds(j, 1), :] = (i + off)[None, :]
        return i

    lax.fori_loop(0, K2, step, jnp.full((R,), N, jnp.int32))


def _sc_stage2(Tf, idx_flat):
    PTS, D = Tf.shape
    O = D // 2
    NW = NC * NS
    ppw = PTS // NW
    ngrp = ppw // 2
    mesh = plsc.VectorSubcoreMesh(core_axis_name="c", subcore_axis_name="s")

    @functools.partial(
        pl.kernel,
        out_type=jax.ShapeDtypeStruct((PTS, 8 * O), jnp.float32),
        mesh=mesh,
        scratch_types=[
            pltpu.VMEM((ppw * K2,), jnp.int32),
            pltpu.VMEM((K2, D), jnp.float32),
            pltpu.VMEM((K2, D), jnp.float32),
            pltpu.VMEM((8 * O,), jnp.float32),
            pltpu.SemaphoreType.DMA,
            pltpu.SemaphoreType.DMA,
        ],
    )
    def sc_k(T_hbm, idx_hbm, out_hbm, idx_v, rv0, rv1, orow_v, sem0, sem1):
        wid = lax.axis_index("s") * NC + lax.axis_index("c")
        base = wid * ppw
        pltpu.sync_copy(idx_hbm.at[pl.ds(base * K2, ppw * K2)], idx_v)

        def issue(p, rv, sem):
            pltpu.async_copy(T_hbm.at[idx_v.at[pl.ds(p * K2, K2)]], rv, sem)

        def drain(rv, sem):
            pltpu.make_async_copy(T_hbm.at[pl.ds(0, K2)], rv, sem).wait()

        def compute(p, rv):
            for c in range(D // 16):
                s_half = c < (O // 16)
                hi = K1 if s_half else K2
                sl = pl.ds(c * 16, 16)
                v = rv[0, sl]
                acc_s = v
                acc_q = v * v
                acc_mx = v
                acc_mn = v
                for r in range(1, hi):
                    v = rv[r, sl]
                    acc_s = acc_s + v
                    acc_q = acc_q + v * v
                    acc_mx = jnp.maximum(acc_mx, v)
                    acc_mn = jnp.minimum(acc_mn, v)
                half = 0 if s_half else 4 * O
                cl = c if s_half else c - O // 16
                orow_v[pl.ds(half + cl * 16, 16)] = acc_s
                orow_v[pl.ds(half + O + cl * 16, 16)] = acc_q
                orow_v[pl.ds(half + 2 * O + cl * 16, 16)] = acc_mx
                orow_v[pl.ds(half + 3 * O + cl * 16, 16)] = acc_mn
            pltpu.sync_copy(orow_v, out_hbm.at[base + p])

        issue(0, rv0, sem0)

        def group(g, carry):
            p0 = 2 * g
            issue(p0 + 1, rv1, sem1)
            drain(rv0, sem0)
            compute(p0, rv0)

            @pl.when(g + 1 < ngrp)
            def _():
                issue(p0 + 2, rv0, sem0)

            drain(rv1, sem1)
            compute(p0 + 1, rv1)
            return carry

        lax.fori_loop(0, ngrp, group, 0)

    return sc_k(Tf, idx_flat)


def _stage3a_body(st_ref, Q_ref, part_ref):
    O = Q_ref.shape[1] // 2
    st = st_ref[...]
    Qs = Q_ref[:, :O]
    Ql = Q_ref[:, O:]
    sum_s, ssq_s = st[:, 0:O], st[:, O:2 * O]
    sum_l, ssq_l = st[:, 4 * O:5 * O], st[:, 5 * O:6 * O]
    tot_s = jnp.sum(sum_s + K1 * Qs, axis=0)
    tot2_s = jnp.sum(ssq_s + 2.0 * Qs * sum_s + K1 * Qs * Qs, axis=0)
    tot_l = jnp.sum(sum_l + K2 * Ql, axis=0)
    tot2_l = jnp.sum(ssq_l + 2.0 * Ql * sum_l + K2 * Ql * Ql, axis=0)
    part_ref[0, 0] = jnp.concatenate([tot_s, tot2_s, tot_l, tot2_l])


def _stage3b_body(st_ref, Q_ref, part_ref, Wf_ref, gs_ref, bs_ref, gl_ref,
                  bl_ref, yf_ref, fpart_ref, *, pts):
    O = Q_ref.shape[1] // 2
    st = st_ref[...]
    Qs = Q_ref[:, :O]
    Ql = Q_ref[:, O:]
    tot = jnp.sum(part_ref[...], axis=0)[0]

    def conv_half(mx_g, mn_g, Q, t, t2, gamma, beta, k):
        cnt = pts * k
        mean = t / cnt
        var = t2 / cnt - mean * mean
        a = gamma * lax.rsqrt(var + EPS)
        c = beta - mean * a
        sel = jnp.where(a >= 0, mx_g, mn_g)
        y = a[None, :] * (sel + Q) + c[None, :]
        return jnp.where(y >= 0, y, SLOPE * y)

    ys = conv_half(st[:, 2 * O:3 * O], st[:, 3 * O:4 * O], Qs,
                   tot[0:O], tot[O:2 * O], gs_ref[0], bs_ref[0], K1)
    yl = conv_half(st[:, 6 * O:7 * O], st[:, 7 * O:8 * O], Ql,
                   tot[2 * O:3 * O], tot[3 * O:4 * O], gl_ref[0], bl_ref[0], K2)
    ycat = jnp.concatenate([ys, yl], axis=1)
    yf = lax.dot_general(ycat, Wf_ref[...], (((1,), (1,)), ((), ())),
                         preferred_element_type=jnp.float32)
    yf_ref[...] = yf
    fpart_ref[0, 0] = jnp.concatenate(
        [jnp.sum(yf, axis=0), jnp.sum(yf * yf, axis=0)])


def _stage3c_body(yf_ref, fpart_ref, gf_ref, bf_ref, out_ref):
    PTS = yf_ref.shape[0]
    yf = yf_ref[...]
    tot = jnp.sum(fpart_ref[...], axis=0)[0]
    O = yf.shape[1]
    m = tot[:O] / PTS
    v = tot[O:] / PTS - m * m
    a = gf_ref[0] * lax.rsqrt(v + EPS)
    c = bf_ref[0] - m * a
    y = a[None, :] * yf + c[None, :]
    out_ref[...] = jnp.where(y >= 0, y, SLOPE * y)


def kernel(x, W_s, gamma_s, beta_s, W_l, gamma_l, beta_l, W_f, gamma_f, beta_f):
    B, C, N = x.shape
    O = W_s.shape[0]
    R = ROWS
    nR = N // R
    PTS = B * N

    T, Qc, idx4 = pl.pallas_call(
        _stage1_body,
        grid=(B, nR),
        in_specs=[
            pl.BlockSpec((1, C, N), lambda b, r: (b, 0, 0)),
            pl.BlockSpec((1, C, R), lambda b, r: (b, 0, r)),
            pl.BlockSpec((O, 2 * C), lambda b, r: (0, 0)),
            pl.BlockSpec((O, 2 * C), lambda b, r: (0, 0)),
        ],
        out_specs=[
            pl.BlockSpec((1, R, 2 * O), lambda b, r: (b, r, 0)),
            pl.BlockSpec((1, R, 2 * O), lambda b, r: (b, r, 0)),
            pl.BlockSpec((1, 1, K2, R), lambda b, r: (b, r, 0, 0)),
        ],
        out_shape=[
            jax.ShapeDtypeStruct((B, N, 2 * O), jnp.float32),
            jax.ShapeDtypeStruct((B, N, 2 * O), jnp.float32),
            jax.ShapeDtypeStruct((B, nR, K2, R), jnp.int32),
        ],
        scratch_shapes=[pltpu.VMEM((N, R), jnp.float32)],
    )(x, x, W_s, W_l)

    idx_flat = idx4.transpose(0, 1, 3, 2).reshape(-1)
    Qf = Qc.reshape(PTS, 2 * O)

    stats = _sc_stage2(T.reshape(PTS, 2 * O), idx_flat)

    CH = 1024
    nch = PTS // CH
    st_spec = pl.BlockSpec((CH, 8 * O), lambda i: (i, 0))
    q_spec = pl.BlockSpec((CH, 2 * O), lambda i: (i, 0))
    part_spec = pl.BlockSpec((1, 1, 4 * O), lambda i: (i, 0, 0))
    full = lambda shape: pl.BlockSpec(shape, lambda i: tuple(0 for _ in shape))

    part = pl.pallas_call(
        _stage3a_body,
        grid=(nch,),
        in_specs=[st_spec, q_spec],
        out_specs=part_spec,
        out_shape=jax.ShapeDtypeStruct((nch, 1, 4 * O), jnp.float32),
    )(stats, Qf)

    yf, fpart = pl.pallas_call(
        functools.partial(_stage3b_body, pts=PTS),
        grid=(nch,),
        in_specs=[st_spec, q_spec, full((nch, 1, 4 * O)), full((O, 2 * O)),
                  full((1, O)), full((1, O)), full((1, O)), full((1, O))],
        out_specs=[pl.BlockSpec((CH, O), lambda i: (i, 0)),
                   pl.BlockSpec((1, 1, 2 * O), lambda i: (i, 0, 0))],
        out_shape=[jax.ShapeDtypeStruct((PTS, O), jnp.float32),
                   jax.ShapeDtypeStruct((nch, 1, 2 * O), jnp.float32)],
    )(stats, Qf, part, W_f,
      gamma_s.reshape(1, O), beta_s.reshape(1, O),
      gamma_l.reshape(1, O), beta_l.reshape(1, O))

    rows = pl.pallas_call(
        _stage3c_body,
        out_shape=jax.ShapeDtypeStruct((PTS, O), jnp.float32),
    )(yf, fpart, gamma_f.reshape(1, O), beta_f.reshape(1, O))

    return rows.reshape(B, N, O).transpose(0, 2, 1)


# speedup vs baseline: 2.5295x; 1.0999x over previous
import functools

import jax
import jax.numpy as jnp
from jax import lax
from jax.experimental import pallas as pl
from jax.experimental.pallas import tpu as pltpu
from jax.experimental.pallas import tpu_sc as plsc

EPS = 1e-5
SLOPE = 0.2
K1 = 20
K2 = 40
ROWS = 512
NC = 2
NS = 16


def _tree_reduce(a, op):
    s = a.shape[0]
    while s > 1:
        h = s // 2
        a = op(a[:h], a[h:])
        s = h
    return a[0]


def _stage1_body(xf_ref, xr_ref, Ws_ref, Wl_ref, T_ref, Q_ref,
                 idx_ref, vals_ref):
    C = xf_ref.shape[1]
    N = xf_ref.shape[2]
    R = xr_ref.shape[2]
    xb = xf_ref[0]
    xr = xr_ref[0]

    dn = (((0,), (1,)), ((), ()))
    A_s = Ws_ref[:, :C]
    A_l = Wl_ref[:, :C]
    B_s = Ws_ref[:, C:] - A_s
    B_l = Wl_ref[:, C:] - A_l
    Ts = lax.dot_general(xr, A_s, dn, preferred_element_type=jnp.float32)
    Tl = lax.dot_general(xr, A_l, dn, preferred_element_type=jnp.float32)
    Qs = lax.dot_general(xr, B_s, dn, preferred_element_type=jnp.float32)
    Ql = lax.dot_general(xr, B_l, dn, preferred_element_type=jnp.float32)
    T_ref[0] = jnp.concatenate([Ts, Tl], axis=1)
    Q_ref[0] = jnp.concatenate([Qs, Ql], axis=1)

    inner = -2.0 * lax.dot_general(xb, xr, (((0,), (0,)), ((), ())),
                                   preferred_element_type=jnp.float32)
    xx_full = jnp.sum(xb * xb, axis=0)
    xx_r = jnp.sum(xr * xr, axis=0)
    vals_ref[...] = -xx_r[None, :] - inner - xx_full[:, None]

    iota = lax.broadcasted_iota(jnp.int32, (N, R), 0)

    def step(j, i_prev):
        vals = jnp.where(iota == i_prev[None, :], -jnp.inf, vals_ref[...])
        vals_ref[...] = vals
        v, ii = vals, iota
        s = N
        while s > 1:
            h = s // 2
            take_hi = v[h:] > v[:h]
            v = jnp.where(take_hi, v[h:], v[:h])
            ii = jnp.where(take_hi, ii[h:], ii[:h])
            s = h
        i = ii[0]
        idx_ref[0, 0, pl.ds(j, 1), :] = i[None, :]
        return i

    lax.fori_loop(0, K2, step, jnp.full((R,), N, jnp.int32))


def _sc_stage2(Tf, idx_flat):
    PTS, D = Tf.shape
    O = D // 2
    NW = NC * NS
    ppw = PTS // NW
    ngrp = ppw // 2
    mesh = plsc.VectorSubcoreMesh(core_axis_name="c", subcore_axis_name="s")

    @functools.partial(
        pl.kernel,
        out_type=jax.ShapeDtypeStruct((PTS, 8 * O), jnp.float32),
        mesh=mesh,
        scratch_types=[
            pltpu.VMEM((ppw * K2,), jnp.int32),
            pltpu.VMEM((K2, D), jnp.float32),
            pltpu.VMEM((K2, D), jnp.float32),
            pltpu.VMEM((8 * O,), jnp.float32),
            pltpu.SemaphoreType.DMA,
            pltpu.SemaphoreType.DMA,
        ],
    )
    def sc_k(T_hbm, idx_hbm, out_hbm, idx_v, rv0, rv1, orow_v, sem0, sem1):
        wid = lax.axis_index("s") * NC + lax.axis_index("c")
        base = wid * ppw
        pltpu.sync_copy(idx_hbm.at[pl.ds(base * K2, ppw * K2)], idx_v)

        def issue(p, rv, sem):
            pltpu.async_copy(T_hbm.at[idx_v.at[pl.ds(p * K2, K2)]], rv, sem)

        def drain(rv, sem):
            pltpu.make_async_copy(T_hbm.at[pl.ds(0, K2)], rv, sem).wait()

        def compute(p, rv):
            for c in range(D // 16):
                s_half = c < (O // 16)
                hi = K1 if s_half else K2
                sl = pl.ds(c * 16, 16)
                v = rv[0, sl]
                acc_s = v
                acc_q = v * v
                acc_mx = v
                acc_mn = v
                for r in range(1, hi):
                    v = rv[r, sl]
                    acc_s = acc_s + v
                    acc_q = acc_q + v * v
                    acc_mx = jnp.maximum(acc_mx, v)
                    acc_mn = jnp.minimum(acc_mn, v)
                half = 0 if s_half else 4 * O
                cl = c if s_half else c - O // 16
                orow_v[pl.ds(half + cl * 16, 16)] = acc_s
                orow_v[pl.ds(half + O + cl * 16, 16)] = acc_q
                orow_v[pl.ds(half + 2 * O + cl * 16, 16)] = acc_mx
                orow_v[pl.ds(half + 3 * O + cl * 16, 16)] = acc_mn
            pltpu.sync_copy(orow_v, out_hbm.at[base + p])

        issue(0, rv0, sem0)

        def group(g, carry):
            p0 = 2 * g
            issue(p0 + 1, rv1, sem1)
            drain(rv0, sem0)
            compute(p0, rv0)

            @pl.when(g + 1 < ngrp)
            def _():
                issue(p0 + 2, rv0, sem0)

            drain(rv1, sem1)
            compute(p0 + 1, rv1)
            return carry

        lax.fori_loop(0, ngrp, group, 0)

    return sc_k(Tf, idx_flat)


def _stage3a_body(st_ref, Q_ref, part_ref):
    O = Q_ref.shape[1] // 2
    st = st_ref[...]
    Qs = Q_ref[:, :O]
    Ql = Q_ref[:, O:]
    sum_s, ssq_s = st[:, 0:O], st[:, O:2 * O]
    sum_l, ssq_l = st[:, 4 * O:5 * O], st[:, 5 * O:6 * O]
    tot_s = jnp.sum(sum_s + K1 * Qs, axis=0)
    tot2_s = jnp.sum(ssq_s + 2.0 * Qs * sum_s + K1 * Qs * Qs, axis=0)
    tot_l = jnp.sum(sum_l + K2 * Ql, axis=0)
    tot2_l = jnp.sum(ssq_l + 2.0 * Ql * sum_l + K2 * Ql * Ql, axis=0)
    part_ref[0, 0] = jnp.concatenate([tot_s, tot2_s, tot_l, tot2_l])


def _stage3b_body(st_ref, Q_ref, part_ref, Wf_ref, gs_ref, bs_ref, gl_ref,
                  bl_ref, yf_ref, fpart_ref, *, pts):
    O = Q_ref.shape[1] // 2
    st = st_ref[...]
    Qs = Q_ref[:, :O]
    Ql = Q_ref[:, O:]
    tot = jnp.sum(part_ref[...], axis=0)[0]

    def conv_half(mx_g, mn_g, Q, t, t2, gamma, beta, k):
        cnt = pts * k
        mean = t / cnt
        var = t2 / cnt - mean * mean
        a = gamma * lax.rsqrt(var + EPS)
        c = beta - mean * a
        sel = jnp.where(a >= 0, mx_g, mn_g)
        y = a[None, :] * (sel + Q) + c[None, :]
        return jnp.where(y >= 0, y, SLOPE * y)

    ys = conv_half(st[:, 2 * O:3 * O], st[:, 3 * O:4 * O], Qs,
                   tot[0:O], tot[O:2 * O], gs_ref[0], bs_ref[0], K1)
    yl = conv_half(st[:, 6 * O:7 * O], st[:, 7 * O:8 * O], Ql,
                   tot[2 * O:3 * O], tot[3 * O:4 * O], gl_ref[0], bl_ref[0], K2)
    ycat = jnp.concatenate([ys, yl], axis=1)
    yf = lax.dot_general(ycat, Wf_ref[...], (((1,), (1,)), ((), ())),
                         preferred_element_type=jnp.float32)
    yf_ref[...] = yf
    fpart_ref[0, 0] = jnp.concatenate(
        [jnp.sum(yf, axis=0), jnp.sum(yf * yf, axis=0)])


def _stage3c_body(yf_ref, fpart_ref, gf_ref, bf_ref, out_ref):
    PTS = yf_ref.shape[0]
    yf = yf_ref[...]
    tot = jnp.sum(fpart_ref[...], axis=0)[0]
    O = yf.shape[1]
    m = tot[:O] / PTS
    v = tot[O:] / PTS - m * m
    a = gf_ref[0] * lax.rsqrt(v + EPS)
    c = bf_ref[0] - m * a
    y = a[None, :] * yf + c[None, :]
    out_ref[...] = jnp.where(y >= 0, y, SLOPE * y)


def kernel(x, W_s, gamma_s, beta_s, W_l, gamma_l, beta_l, W_f, gamma_f, beta_f):
    B, C, N = x.shape
    O = W_s.shape[0]
    R = ROWS
    nR = N // R
    PTS = B * N

    stage1 = pl.pallas_call(
        _stage1_body,
        grid=(nR,),
        in_specs=[
            pl.BlockSpec((1, C, N), lambda r: (0, 0, 0)),
            pl.BlockSpec((1, C, R), lambda r: (0, 0, r)),
            pl.BlockSpec((O, 2 * C), lambda r: (0, 0)),
            pl.BlockSpec((O, 2 * C), lambda r: (0, 0)),
        ],
        out_specs=[
            pl.BlockSpec((1, R, 2 * O), lambda r: (0, r, 0)),
            pl.BlockSpec((1, R, 2 * O), lambda r: (0, r, 0)),
            pl.BlockSpec((1, 1, K2, R), lambda r: (0, r, 0, 0)),
        ],
        out_shape=[
            jax.ShapeDtypeStruct((1, N, 2 * O), jnp.float32),
            jax.ShapeDtypeStruct((1, N, 2 * O), jnp.float32),
            jax.ShapeDtypeStruct((1, nR, K2, R), jnp.int32),
        ],
        scratch_shapes=[pltpu.VMEM((N, R), jnp.float32)],
    )

    stats_parts, q_parts = [], []
    for b in range(B):
        xb = lax.slice_in_dim(x, b, b + 1, axis=0)
        Tb, Qb, idxb = stage1(xb, xb, W_s, W_l)
        idxb_flat = idxb.transpose(0, 1, 3, 2).reshape(-1)
        stats_parts.append(_sc_stage2(Tb.reshape(N, 2 * O), idxb_flat))
        q_parts.append(Qb.reshape(N, 2 * O))
    stats = jnp.concatenate(stats_parts, axis=0)
    Qf = jnp.concatenate(q_parts, axis=0)

    CH = 1024
    nch = PTS // CH
    st_spec = pl.BlockSpec((CH, 8 * O), lambda i: (i, 0))
    q_spec = pl.BlockSpec((CH, 2 * O), lambda i: (i, 0))
    part_spec = pl.BlockSpec((1, 1, 4 * O), lambda i: (i, 0, 0))
    full = lambda shape: pl.BlockSpec(shape, lambda i: tuple(0 for _ in shape))

    part = pl.pallas_call(
        _stage3a_body,
        grid=(nch,),
        in_specs=[st_spec, q_spec],
        out_specs=part_spec,
        out_shape=jax.ShapeDtypeStruct((nch, 1, 4 * O), jnp.float32),
    )(stats, Qf)

    yf, fpart = pl.pallas_call(
        functools.partial(_stage3b_body, pts=PTS),
        grid=(nch,),
        in_specs=[st_spec, q_spec, full((nch, 1, 4 * O)), full((O, 2 * O)),
                  full((1, O)), full((1, O)), full((1, O)), full((1, O))],
        out_specs=[pl.BlockSpec((CH, O), lambda i: (i, 0)),
                   pl.BlockSpec((1, 1, 2 * O), lambda i: (i, 0, 0))],
        out_shape=[jax.ShapeDtypeStruct((PTS, O), jnp.float32),
                   jax.ShapeDtypeStruct((nch, 1, 2 * O), jnp.float32)],
    )(stats, Qf, part, W_f,
      gamma_s.reshape(1, O), beta_s.reshape(1, O),
      gamma_l.reshape(1, O), beta_l.reshape(1, O))

    rows = pl.pallas_call(
        _stage3c_body,
        out_shape=jax.ShapeDtypeStruct((PTS, O), jnp.float32),
    )(yf, fpart, gamma_f.reshape(1, O), beta_f.reshape(1, O))

    return rows.reshape(B, N, O).transpose(0, 2, 1)


# speedup vs baseline: 2.5812x; 1.0204x over previous
import functools

import jax
import jax.numpy as jnp
from jax import lax
from jax.experimental import pallas as pl
from jax.experimental.pallas import tpu as pltpu
from jax.experimental.pallas import tpu_sc as plsc

EPS = 1e-5
SLOPE = 0.2
K1 = 20
K2 = 40
ROWS = 512
NC = 2
NS = 16


def _tree_reduce(a, op):
    s = a.shape[0]
    while s > 1:
        h = s // 2
        a = op(a[:h], a[h:])
        s = h
    return a[0]


def _stage1_body(xf_ref, xr_ref, Ws_ref, Wl_ref, T_ref, Q_ref,
                 idx_ref, vals_ref):
    C = xf_ref.shape[1]
    N = xf_ref.shape[2]
    R = xr_ref.shape[2]
    xb = xf_ref[0]
    xr = xr_ref[0]

    dn = (((0,), (1,)), ((), ()))
    A_s = Ws_ref[:, :C]
    A_l = Wl_ref[:, :C]
    B_s = Ws_ref[:, C:] - A_s
    B_l = Wl_ref[:, C:] - A_l
    Ts = lax.dot_general(xr, A_s, dn, preferred_element_type=jnp.float32)
    Tl = lax.dot_general(xr, A_l, dn, preferred_element_type=jnp.float32)
    Qs = lax.dot_general(xr, B_s, dn, preferred_element_type=jnp.float32)
    Ql = lax.dot_general(xr, B_l, dn, preferred_element_type=jnp.float32)
    T_ref[0] = jnp.concatenate([Ts, Tl], axis=1)
    Q_ref[0] = jnp.concatenate([Qs, Ql], axis=1)

    inner = -2.0 * lax.dot_general(xb, xr, (((0,), (0,)), ((), ())),
                                   preferred_element_type=jnp.float32)
    xx_full = jnp.sum(xb * xb, axis=0)
    xx_r = jnp.sum(xr * xr, axis=0)
    vals_ref[...] = -xx_r[None, :] - inner - xx_full[:, None]

    iota = lax.broadcasted_iota(jnp.int32, (N, R), 0)

    def step(j, i_prev):
        vals = jnp.where(iota == i_prev[None, :], -jnp.inf, vals_ref[...])
        vals_ref[...] = vals
        v, ii = vals, iota
        s = N
        while s > 1:
            h = s // 2
            take_hi = v[h:] > v[:h]
            v = jnp.where(take_hi, v[h:], v[:h])
            ii = jnp.where(take_hi, ii[h:], ii[:h])
            s = h
        i = ii[0]
        idx_ref[0, 0, pl.ds(j, 1), :] = i[None, :]
        return i

    lax.fori_loop(0, K2, step, jnp.full((R,), N, jnp.int32))


def _sc_stage2(Tf, idx_flat):
    PTS, D = Tf.shape
    O = D // 2
    NW = NC * NS
    ppw = PTS // NW
    ngrp = ppw // 2
    mesh = plsc.VectorSubcoreMesh(core_axis_name="c", subcore_axis_name="s")

    @functools.partial(
        pl.kernel,
        out_type=jax.ShapeDtypeStruct((PTS, 8 * O), jnp.float32),
        mesh=mesh,
        scratch_types=[
            pltpu.VMEM((ppw * K2,), jnp.int32),
            pltpu.VMEM((K2, D), jnp.float32),
            pltpu.VMEM((K2, D), jnp.float32),
            pltpu.VMEM((8 * O,), jnp.float32),
            pltpu.SemaphoreType.DMA,
            pltpu.SemaphoreType.DMA,
        ],
    )
    def sc_k(T_hbm, idx_hbm, out_hbm, idx_v, rv0, rv1, orow_v, sem0, sem1):
        wid = lax.axis_index("s") * NC + lax.axis_index("c")
        base = wid * ppw
        pltpu.sync_copy(idx_hbm.at[pl.ds(base * K2, ppw * K2)], idx_v)

        def issue(p, rv, sem):
            pltpu.async_copy(T_hbm.at[idx_v.at[pl.ds(p * K2, K2)]], rv, sem)

        def drain(rv, sem):
            pltpu.make_async_copy(T_hbm.at[pl.ds(0, K2)], rv, sem).wait()

        def compute(p, rv):
            for c in range(D // 16):
                s_half = c < (O // 16)
                hi = K1 if s_half else K2
                sl = pl.ds(c * 16, 16)
                v = rv[0, sl]
                acc_s = v
                acc_q = v * v
                acc_mx = v
                acc_mn = v
                for r in range(1, hi):
                    v = rv[r, sl]
                    acc_s = acc_s + v
                    acc_q = acc_q + v * v
                    acc_mx = jnp.maximum(acc_mx, v)
                    acc_mn = jnp.minimum(acc_mn, v)
                half = 0 if s_half else 4 * O
                cl = c if s_half else c - O // 16
                orow_v[pl.ds(half + cl * 16, 16)] = acc_s
                orow_v[pl.ds(half + O + cl * 16, 16)] = acc_q
                orow_v[pl.ds(half + 2 * O + cl * 16, 16)] = acc_mx
                orow_v[pl.ds(half + 3 * O + cl * 16, 16)] = acc_mn
            pltpu.sync_copy(orow_v, out_hbm.at[base + p])

        issue(0, rv0, sem0)

        def group(g, carry):
            p0 = 2 * g
            issue(p0 + 1, rv1, sem1)
            drain(rv0, sem0)
            compute(p0, rv0)

            @pl.when(g + 1 < ngrp)
            def _():
                issue(p0 + 2, rv0, sem0)

            drain(rv1, sem1)
            compute(p0 + 1, rv1)
            return carry

        lax.fori_loop(0, ngrp, group, 0)

    return sc_k(Tf, idx_flat)


def _stage3_body(st_ref, q_ref, wf_ref, gs_ref, bs_ref, gl_ref, bl_ref,
                 gf_ref, bf_ref, out_ref, acc_ref, yf_ref, *, pts, ch):
    p = pl.program_id(0)
    i = pl.program_id(1)
    O = q_ref.shape[1] // 2

    @pl.when((p == 0) & (i == 0))
    def _init():
        acc_ref[...] = jnp.zeros_like(acc_ref)

    @pl.when(p == 0)
    def _tot():
        st = st_ref[...]
        Qs = q_ref[:, :O]
        Ql = q_ref[:, O:]
        sum_s, ssq_s = st[:, 0:O], st[:, O:2 * O]
        sum_l, ssq_l = st[:, 4 * O:5 * O], st[:, 5 * O:6 * O]
        tot_s = jnp.sum(sum_s + K1 * Qs, axis=0)
        tot2_s = jnp.sum(ssq_s + 2.0 * Qs * sum_s + K1 * Qs * Qs, axis=0)
        tot_l = jnp.sum(sum_l + K2 * Ql, axis=0)
        tot2_l = jnp.sum(ssq_l + 2.0 * Ql * sum_l + K2 * Ql * Ql, axis=0)
        acc_ref[0, :4 * O] = acc_ref[0, :4 * O] + jnp.concatenate(
            [tot_s, tot2_s, tot_l, tot2_l])

    @pl.when(p == 1)
    def _conv():
        st = st_ref[...]
        Qs = q_ref[:, :O]
        Ql = q_ref[:, O:]
        tot = acc_ref[0]

        def conv_half(mx_g, mn_g, Q, t, t2, gamma, beta, k):
            cnt = pts * k
            mean = t / cnt
            var = t2 / cnt - mean * mean
            a = gamma * lax.rsqrt(var + EPS)
            c = beta - mean * a
            sel = jnp.where(a >= 0, mx_g, mn_g)
            y = a[None, :] * (sel + Q) + c[None, :]
            return jnp.where(y >= 0, y, SLOPE * y)

        ys = conv_half(st[:, 2 * O:3 * O], st[:, 3 * O:4 * O], Qs,
                       tot[0:O], tot[O:2 * O], gs_ref[0], bs_ref[0], K1)
        yl = conv_half(st[:, 6 * O:7 * O], st[:, 7 * O:8 * O], Ql,
                       tot[2 * O:3 * O], tot[3 * O:4 * O],
                       gl_ref[0], bl_ref[0], K2)
        ycat = jnp.concatenate([ys, yl], axis=1)
        yf = lax.dot_general(ycat, wf_ref[...], (((1,), (1,)), ((), ())),
                             preferred_element_type=jnp.float32)
        yf_ref[pl.ds(i * ch, ch), :] = yf
        acc_ref[0, 4 * O:6 * O] = acc_ref[0, 4 * O:6 * O] + jnp.concatenate(
            [jnp.sum(yf, axis=0), jnp.sum(yf * yf, axis=0)])

    @pl.when(p == 2)
    def _fuse():
        tot = acc_ref[0]
        m = tot[4 * O:5 * O] / pts
        v = tot[5 * O:6 * O] / pts - m * m
        a = gf_ref[0] * lax.rsqrt(v + EPS)
        c = bf_ref[0] - m * a
        yf = yf_ref[pl.ds(i * ch, ch), :]
        y = a[None, :] * yf + c[None, :]
        out_ref[...] = jnp.where(y >= 0, y, SLOPE * y)


def kernel(x, W_s, gamma_s, beta_s, W_l, gamma_l, beta_l, W_f, gamma_f, beta_f):
    B, C, N = x.shape
    O = W_s.shape[0]
    R = ROWS
    nR = N // R
    PTS = B * N

    stage1 = pl.pallas_call(
        _stage1_body,
        grid=(nR,),
        in_specs=[
            pl.BlockSpec((1, C, N), lambda r: (0, 0, 0)),
            pl.BlockSpec((1, C, R), lambda r: (0, 0, r)),
            pl.BlockSpec((O, 2 * C), lambda r: (0, 0)),
            pl.BlockSpec((O, 2 * C), lambda r: (0, 0)),
        ],
        out_specs=[
            pl.BlockSpec((1, R, 2 * O), lambda r: (0, r, 0)),
            pl.BlockSpec((1, R, 2 * O), lambda r: (0, r, 0)),
            pl.BlockSpec((1, 1, K2, R), lambda r: (0, r, 0, 0)),
        ],
        out_shape=[
            jax.ShapeDtypeStruct((1, N, 2 * O), jnp.float32),
            jax.ShapeDtypeStruct((1, N, 2 * O), jnp.float32),
            jax.ShapeDtypeStruct((1, nR, K2, R), jnp.int32),
        ],
        scratch_shapes=[pltpu.VMEM((N, R), jnp.float32)],
    )

    stats_parts, q_parts = [], []
    for b in range(B):
        xb = lax.slice_in_dim(x, b, b + 1, axis=0)
        Tb, Qb, idxb = stage1(xb, xb, W_s, W_l)
        idxb_flat = idxb.transpose(0, 1, 3, 2).reshape(-1)
        stats_parts.append(_sc_stage2(Tb.reshape(N, 2 * O), idxb_flat))
        q_parts.append(Qb.reshape(N, 2 * O))
    stats = jnp.concatenate(stats_parts, axis=0)
    Qf = jnp.concatenate(q_parts, axis=0)

    CH = 1024
    nch = PTS // CH
    st_spec = pl.BlockSpec((CH, 8 * O), lambda p, i: (jnp.where(p < 2, i, 0), 0))
    q_spec = pl.BlockSpec((CH, 2 * O), lambda p, i: (jnp.where(p < 2, i, 0), 0))
    full = lambda shape: pl.BlockSpec(shape, lambda p, i: tuple(0 for _ in shape))

    rows = pl.pallas_call(
        functools.partial(_stage3_body, pts=PTS, ch=CH),
        grid=(3, nch),
        in_specs=[st_spec, q_spec, full((O, 2 * O)),
                  full((1, O)), full((1, O)), full((1, O)), full((1, O)),
                  full((1, O)), full((1, O))],
        out_specs=pl.BlockSpec((CH, O), lambda p, i: (i, 0)),
        out_shape=jax.ShapeDtypeStruct((PTS, O), jnp.float32),
        scratch_shapes=[pltpu.VMEM((1, 6 * O), jnp.float32),
                        pltpu.VMEM((PTS, O), jnp.float32)],
    )(stats, Qf, W_f,
      gamma_s.reshape(1, O), beta_s.reshape(1, O),
      gamma_l.reshape(1, O), beta_l.reshape(1, O),
      gamma_f.reshape(1, O), beta_f.reshape(1, O))

    return rows.reshape(B, N, O).transpose(0, 2, 1)


# speedup vs baseline: 2.5848x; 1.0014x over previous
import functools

import jax
import jax.numpy as jnp
from jax import lax
from jax.experimental import pallas as pl
from jax.experimental.pallas import tpu as pltpu
from jax.experimental.pallas import tpu_sc as plsc

EPS = 1e-5
SLOPE = 0.2
K1 = 20
K2 = 40
ROWS = 512
NC = 2
NS = 16


def _tree_reduce(a, op):
    s = a.shape[0]
    while s > 1:
        h = s // 2
        a = op(a[:h], a[h:])
        s = h
    return a[0]


def _stage1_body(xf_ref, xr_ref, Ws_ref, Wl_ref, T_ref, Q_ref,
                 idx_ref, vals_ref):
    C = xf_ref.shape[1]
    N = xf_ref.shape[2]
    R = xr_ref.shape[2]
    xb = xf_ref[0]
    xr = xr_ref[0]

    dn = (((0,), (1,)), ((), ()))
    A_s = Ws_ref[:, :C]
    A_l = Wl_ref[:, :C]
    B_s = Ws_ref[:, C:] - A_s
    B_l = Wl_ref[:, C:] - A_l
    Ts = lax.dot_general(xr, A_s, dn, preferred_element_type=jnp.float32)
    Tl = lax.dot_general(xr, A_l, dn, preferred_element_type=jnp.float32)
    Qs = lax.dot_general(xr, B_s, dn, preferred_element_type=jnp.float32)
    Ql = lax.dot_general(xr, B_l, dn, preferred_element_type=jnp.float32)
    T_ref[0] = jnp.concatenate([Ts, Tl], axis=1)
    Q_ref[0] = jnp.concatenate([Qs, Ql], axis=1)

    inner = -2.0 * lax.dot_general(xb, xr, (((0,), (0,)), ((), ())),
                                   preferred_element_type=jnp.float32)
    xx_full = jnp.sum(xb * xb, axis=0)
    xx_r = jnp.sum(xr * xr, axis=0)
    vals_ref[...] = -xx_r[None, :] - inner - xx_full[:, None]

    iota = lax.broadcasted_iota(jnp.int32, (N, R), 0)

    def step(j, i_prev):
        vals = jnp.where(iota == i_prev[None, :], -jnp.inf, vals_ref[...])
        vals_ref[...] = vals
        v, ii = vals, iota
        s = N
        while s > 1:
            h = s // 2
            take_hi = v[h:] > v[:h]
            v = jnp.where(take_hi, v[h:], v[:h])
            ii = jnp.where(take_hi, ii[h:], ii[:h])
            s = h
        i = ii[0]
        idx_ref[0, 0, pl.ds(j, 1), :] = i[None, :]
        return i

    lax.fori_loop(0, K2, step, jnp.full((R,), N, jnp.int32))


def _sc_stage2(Tf, idx_flat):
    PTS, D = Tf.shape
    O = D // 2
    NW = NC * NS
    ppw = PTS // NW
    ngrp = ppw // 2
    mesh = plsc.VectorSubcoreMesh(core_axis_name="c", subcore_axis_name="s")

    @functools.partial(
        pl.kernel,
        out_type=jax.ShapeDtypeStruct((PTS, 8 * O), jnp.float32),
        mesh=mesh,
        scratch_types=[
            pltpu.VMEM((ppw * K2,), jnp.int32),
            pltpu.VMEM((K2, D), jnp.float32),
            pltpu.VMEM((K2, D), jnp.float32),
            pltpu.VMEM((8 * O,), jnp.float32),
            pltpu.VMEM((8 * O,), jnp.float32),
            pltpu.SemaphoreType.DMA,
            pltpu.SemaphoreType.DMA,
            pltpu.SemaphoreType.DMA,
            pltpu.SemaphoreType.DMA,
        ],
    )
    def sc_k(T_hbm, idx_hbm, out_hbm, idx_v, rv0, rv1, orow0, orow1,
             sem0, sem1, st0, st1):
        wid = lax.axis_index("s") * NC + lax.axis_index("c")
        base = wid * ppw
        pltpu.sync_copy(idx_hbm.at[pl.ds(base * K2, ppw * K2)], idx_v)

        def issue(p, rv, sem):
            pltpu.async_copy(T_hbm.at[idx_v.at[pl.ds(p * K2, K2)]], rv, sem)

        def drain(rv, sem):
            pltpu.make_async_copy(T_hbm.at[pl.ds(0, K2)], rv, sem).wait()

        def compute(p, rv, orow_v, st):
            for c in range(D // 16):
                s_half = c < (O // 16)
                hi = K1 if s_half else K2
                sl = pl.ds(c * 16, 16)
                v = rv[0, sl]
                acc_s = v
                acc_q = v * v
                acc_mx = v
                acc_mn = v
                for r in range(1, hi):
                    v = rv[r, sl]
                    acc_s = acc_s + v
                    acc_q = acc_q + v * v
                    acc_mx = jnp.maximum(acc_mx, v)
                    acc_mn = jnp.minimum(acc_mn, v)
                half = 0 if s_half else 4 * O
                cl = c if s_half else c - O // 16
                orow_v[pl.ds(half + cl * 16, 16)] = acc_s
                orow_v[pl.ds(half + O + cl * 16, 16)] = acc_q
                orow_v[pl.ds(half + 2 * O + cl * 16, 16)] = acc_mx
                orow_v[pl.ds(half + 3 * O + cl * 16, 16)] = acc_mn
            pltpu.async_copy(orow_v, out_hbm.at[base + p], st)

        def drain_store(orow_v, st):
            pltpu.make_async_copy(orow_v, out_hbm.at[base], st).wait()

        issue(0, rv0, sem0)

        def group(g, carry):
            p0 = 2 * g
            issue(p0 + 1, rv1, sem1)
            drain(rv0, sem0)

            @pl.when(g > 0)
            def _():
                drain_store(orow0, st0)

            compute(p0, rv0, orow0, st0)

            @pl.when(g + 1 < ngrp)
            def _():
                issue(p0 + 2, rv0, sem0)

            drain(rv1, sem1)

            @pl.when(g > 0)
            def _():
                drain_store(orow1, st1)

            compute(p0 + 1, rv1, orow1, st1)
            return carry

        lax.fori_loop(0, ngrp, group, 0)
        drain_store(orow0, st0)
        drain_store(orow1, st1)

    return sc_k(Tf, idx_flat)


def _stage3_body(st_ref, q_ref, wf_ref, gs_ref, bs_ref, gl_ref, bl_ref,
                 gf_ref, bf_ref, out_ref, acc_ref, yf_ref, *, pts, ch):
    p = pl.program_id(0)
    i = pl.program_id(1)
    O = q_ref.shape[1] // 2

    @pl.when((p == 0) & (i == 0))
    def _init():
        acc_ref[...] = jnp.zeros_like(acc_ref)

    @pl.when(p == 0)
    def _tot():
        st = st_ref[...]
        Qs = q_ref[:, :O]
        Ql = q_ref[:, O:]
        sum_s, ssq_s = st[:, 0:O], st[:, O:2 * O]
        sum_l, ssq_l = st[:, 4 * O:5 * O], st[:, 5 * O:6 * O]
        tot_s = jnp.sum(sum_s + K1 * Qs, axis=0)
        tot2_s = jnp.sum(ssq_s + 2.0 * Qs * sum_s + K1 * Qs * Qs, axis=0)
        tot_l = jnp.sum(sum_l + K2 * Ql, axis=0)
        tot2_l = jnp.sum(ssq_l + 2.0 * Ql * sum_l + K2 * Ql * Ql, axis=0)
        acc_ref[0, :4 * O] = acc_ref[0, :4 * O] + jnp.concatenate(
            [tot_s, tot2_s, tot_l, tot2_l])

    @pl.when(p == 1)
    def _conv():
        st = st_ref[...]
        Qs = q_ref[:, :O]
        Ql = q_ref[:, O:]
        tot = acc_ref[0]

        def conv_half(mx_g, mn_g, Q, t, t2, gamma, beta, k):
            cnt = pts * k
            mean = t / cnt
            var = t2 / cnt - mean * mean
            a = gamma * lax.rsqrt(var + EPS)
            c = beta - mean * a
            sel = jnp.where(a >= 0, mx_g, mn_g)
            y = a[None, :] * (sel + Q) + c[None, :]
            return jnp.where(y >= 0, y, SLOPE * y)

        ys = conv_half(st[:, 2 * O:3 * O], st[:, 3 * O:4 * O], Qs,
                       tot[0:O], tot[O:2 * O], gs_ref[0], bs_ref[0], K1)
        yl = conv_half(st[:, 6 * O:7 * O], st[:, 7 * O:8 * O], Ql,
                       tot[2 * O:3 * O], tot[3 * O:4 * O],
                       gl_ref[0], bl_ref[0], K2)
        ycat = jnp.concatenate([ys, yl], axis=1)
        yf = lax.dot_general(ycat, wf_ref[...], (((1,), (1,)), ((), ())),
                             preferred_element_type=jnp.float32)
        yf_ref[pl.ds(i * ch, ch), :] = yf
        acc_ref[0, 4 * O:6 * O] = acc_ref[0, 4 * O:6 * O] + jnp.concatenate(
            [jnp.sum(yf, axis=0), jnp.sum(yf * yf, axis=0)])

    @pl.when(p == 2)
    def _fuse():
        tot = acc_ref[0]
        m = tot[4 * O:5 * O] / pts
        v = tot[5 * O:6 * O] / pts - m * m
        a = gf_ref[0] * lax.rsqrt(v + EPS)
        c = bf_ref[0] - m * a
        yf = yf_ref[pl.ds(i * ch, ch), :]
        y = a[None, :] * yf + c[None, :]
        out_ref[...] = jnp.where(y >= 0, y, SLOPE * y)


def kernel(x, W_s, gamma_s, beta_s, W_l, gamma_l, beta_l, W_f, gamma_f, beta_f):
    B, C, N = x.shape
    O = W_s.shape[0]
    R = ROWS
    nR = N // R
    PTS = B * N

    stage1 = pl.pallas_call(
        _stage1_body,
        grid=(nR,),
        in_specs=[
            pl.BlockSpec((1, C, N), lambda r: (0, 0, 0)),
            pl.BlockSpec((1, C, R), lambda r: (0, 0, r)),
            pl.BlockSpec((O, 2 * C), lambda r: (0, 0)),
            pl.BlockSpec((O, 2 * C), lambda r: (0, 0)),
        ],
        out_specs=[
            pl.BlockSpec((1, R, 2 * O), lambda r: (0, r, 0)),
            pl.BlockSpec((1, R, 2 * O), lambda r: (0, r, 0)),
            pl.BlockSpec((1, 1, K2, R), lambda r: (0, r, 0, 0)),
        ],
        out_shape=[
            jax.ShapeDtypeStruct((1, N, 2 * O), jnp.float32),
            jax.ShapeDtypeStruct((1, N, 2 * O), jnp.float32),
            jax.ShapeDtypeStruct((1, nR, K2, R), jnp.int32),
        ],
        scratch_shapes=[pltpu.VMEM((N, R), jnp.float32)],
    )

    stats_parts, q_parts = [], []
    for b in range(B):
        xb = lax.slice_in_dim(x, b, b + 1, axis=0)
        Tb, Qb, idxb = stage1(xb, xb, W_s, W_l)
        idxb_flat = idxb.transpose(0, 1, 3, 2).reshape(-1)
        stats_parts.append(_sc_stage2(Tb.reshape(N, 2 * O), idxb_flat))
        q_parts.append(Qb.reshape(N, 2 * O))
    stats = jnp.concatenate(stats_parts, axis=0)
    Qf = jnp.concatenate(q_parts, axis=0)

    CH = 1024
    nch = PTS // CH
    st_spec = pl.BlockSpec((CH, 8 * O), lambda p, i: (jnp.where(p < 2, i, 0), 0))
    q_spec = pl.BlockSpec((CH, 2 * O), lambda p, i: (jnp.where(p < 2, i, 0), 0))
    full = lambda shape: pl.BlockSpec(shape, lambda p, i: tuple(0 for _ in shape))

    rows = pl.pallas_call(
        functools.partial(_stage3_body, pts=PTS, ch=CH),
        grid=(3, nch),
        in_specs=[st_spec, q_spec, full((O, 2 * O)),
                  full((1, O)), full((1, O)), full((1, O)), full((1, O)),
                  full((1, O)), full((1, O))],
        out_specs=pl.BlockSpec((CH, O), lambda p, i: (i, 0)),
        out_shape=jax.ShapeDtypeStruct((PTS, O), jnp.float32),
        scratch_shapes=[pltpu.VMEM((1, 6 * O), jnp.float32),
                        pltpu.VMEM((PTS, O), jnp.float32)],
    )(stats, Qf, W_f,
      gamma_s.reshape(1, O), beta_s.reshape(1, O),
      gamma_l.reshape(1, O), beta_l.reshape(1, O),
      gamma_f.reshape(1, O), beta_f.reshape(1, O))

    return rows.reshape(B, N, O).transpose(0, 2, 1)


# speedup vs baseline: 2.5851x; 1.0001x over previous
import functools

import jax
import jax.numpy as jnp
from jax import lax
from jax.experimental import pallas as pl
from jax.experimental.pallas import tpu as pltpu
from jax.experimental.pallas import tpu_sc as plsc

EPS = 1e-5
SLOPE = 0.2
K1 = 20
K2 = 40
ROWS = 1024
NC = 2
NS = 16


def _tree_reduce(a, op):
    s = a.shape[0]
    while s > 1:
        h = s // 2
        a = op(a[:h], a[h:])
        s = h
    return a[0]


def _stage1_body(xf_ref, xr_ref, Ws_ref, Wl_ref, T_ref, Q_ref,
                 idx_ref, vals_ref):
    C = xf_ref.shape[1]
    N = xf_ref.shape[2]
    R = xr_ref.shape[2]
    xb = xf_ref[0]
    xr = xr_ref[0]

    dn = (((0,), (1,)), ((), ()))
    A_s = Ws_ref[:, :C]
    A_l = Wl_ref[:, :C]
    B_s = Ws_ref[:, C:] - A_s
    B_l = Wl_ref[:, C:] - A_l
    Ts = lax.dot_general(xr, A_s, dn, preferred_element_type=jnp.float32)
    Tl = lax.dot_general(xr, A_l, dn, preferred_element_type=jnp.float32)
    Qs = lax.dot_general(xr, B_s, dn, preferred_element_type=jnp.float32)
    Ql = lax.dot_general(xr, B_l, dn, preferred_element_type=jnp.float32)
    T_ref[0] = jnp.concatenate([Ts, Tl], axis=1)
    Q_ref[0] = jnp.concatenate([Qs, Ql], axis=1)

    inner = -2.0 * lax.dot_general(xb, xr, (((0,), (0,)), ((), ())),
                                   preferred_element_type=jnp.float32)
    xx_full = jnp.sum(xb * xb, axis=0)
    xx_r = jnp.sum(xr * xr, axis=0)
    vals_ref[...] = -xx_r[None, :] - inner - xx_full[:, None]

    iota = lax.broadcasted_iota(jnp.int32, (N, R), 0)

    def step(j, i_prev):
        vals = jnp.where(iota == i_prev[None, :], -jnp.inf, vals_ref[...])
        vals_ref[...] = vals
        v, ii = vals, iota
        s = N
        while s > 1:
            h = s // 2
            take_hi = v[h:] > v[:h]
            v = jnp.where(take_hi, v[h:], v[:h])
            ii = jnp.where(take_hi, ii[h:], ii[:h])
            s = h
        i = ii[0]
        idx_ref[0, 0, pl.ds(j, 1), :] = i[None, :]
        return i

    lax.fori_loop(0, K2, step, jnp.full((R,), N, jnp.int32))


def _sc_stage2(Tf, idx_flat):
    PTS, D = Tf.shape
    O = D // 2
    NW = NC * NS
    ppw = PTS // NW
    ngrp = ppw // 2
    mesh = plsc.VectorSubcoreMesh(core_axis_name="c", subcore_axis_name="s")

    @functools.partial(
        pl.kernel,
        out_type=jax.ShapeDtypeStruct((PTS, 8 * O), jnp.float32),
        mesh=mesh,
        scratch_types=[
            pltpu.VMEM((ppw * K2,), jnp.int32),
            pltpu.VMEM((K2, D), jnp.float32),
            pltpu.VMEM((K2, D), jnp.float32),
            pltpu.VMEM((8 * O,), jnp.float32),
            pltpu.VMEM((8 * O,), jnp.float32),
            pltpu.SemaphoreType.DMA,
            pltpu.SemaphoreType.DMA,
            pltpu.SemaphoreType.DMA,
            pltpu.SemaphoreType.DMA,
        ],
    )
    def sc_k(T_hbm, idx_hbm, out_hbm, idx_v, rv0, rv1, orow0, orow1,
             sem0, sem1, st0, st1):
        wid = lax.axis_index("s") * NC + lax.axis_index("c")
        base = wid * ppw
        pltpu.sync_copy(idx_hbm.at[pl.ds(base * K2, ppw * K2)], idx_v)

        def issue(p, rv, sem):
            pltpu.async_copy(T_hbm.at[idx_v.at[pl.ds(p * K2, K2)]], rv, sem)

        def drain(rv, sem):
            pltpu.make_async_copy(T_hbm.at[pl.ds(0, K2)], rv, sem).wait()

        def compute(p, rv, orow_v, st):
            for c in range(D // 16):
                s_half = c < (O // 16)
                hi = K1 if s_half else K2
                sl = pl.ds(c * 16, 16)
                v = rv[0, sl]
                acc_s = v
                acc_q = v * v
                acc_mx = v
                acc_mn = v
                for r in range(1, hi):
                    v = rv[r, sl]
                    acc_s = acc_s + v
                    acc_q = acc_q + v * v
                    acc_mx = jnp.maximum(acc_mx, v)
                    acc_mn = jnp.minimum(acc_mn, v)
                half = 0 if s_half else 4 * O
                cl = c if s_half else c - O // 16
                orow_v[pl.ds(half + cl * 16, 16)] = acc_s
                orow_v[pl.ds(half + O + cl * 16, 16)] = acc_q
                orow_v[pl.ds(half + 2 * O + cl * 16, 16)] = acc_mx
                orow_v[pl.ds(half + 3 * O + cl * 16, 16)] = acc_mn
            pltpu.async_copy(orow_v, out_hbm.at[base + p], st)

        def drain_store(orow_v, st):
            pltpu.make_async_copy(orow_v, out_hbm.at[base], st).wait()

        issue(0, rv0, sem0)

        def group(g, carry):
            p0 = 2 * g
            issue(p0 + 1, rv1, sem1)
            drain(rv0, sem0)

            @pl.when(g > 0)
            def _():
                drain_store(orow0, st0)

            compute(p0, rv0, orow0, st0)

            @pl.when(g + 1 < ngrp)
            def _():
                issue(p0 + 2, rv0, sem0)

            drain(rv1, sem1)

            @pl.when(g > 0)
            def _():
                drain_store(orow1, st1)

            compute(p0 + 1, rv1, orow1, st1)
            return carry

        lax.fori_loop(0, ngrp, group, 0)
        drain_store(orow0, st0)
        drain_store(orow1, st1)

    return sc_k(Tf, idx_flat)


def _stage3_body(st_ref, q_ref, wf_ref, gs_ref, bs_ref, gl_ref, bl_ref,
                 gf_ref, bf_ref, out_ref, acc_ref, yf_ref, *, pts, ch):
    p = pl.program_id(0)
    i = pl.program_id(1)
    O = q_ref.shape[1] // 2

    @pl.when((p == 0) & (i == 0))
    def _init():
        acc_ref[...] = jnp.zeros_like(acc_ref)

    @pl.when(p == 0)
    def _tot():
        st = st_ref[...]
        Qs = q_ref[:, :O]
        Ql = q_ref[:, O:]
        sum_s, ssq_s = st[:, 0:O], st[:, O:2 * O]
        sum_l, ssq_l = st[:, 4 * O:5 * O], st[:, 5 * O:6 * O]
        tot_s = jnp.sum(sum_s + K1 * Qs, axis=0)
        tot2_s = jnp.sum(ssq_s + 2.0 * Qs * sum_s + K1 * Qs * Qs, axis=0)
        tot_l = jnp.sum(sum_l + K2 * Ql, axis=0)
        tot2_l = jnp.sum(ssq_l + 2.0 * Ql * sum_l + K2 * Ql * Ql, axis=0)
        acc_ref[0, :4 * O] = acc_ref[0, :4 * O] + jnp.concatenate(
            [tot_s, tot2_s, tot_l, tot2_l])

    @pl.when(p == 1)
    def _conv():
        st = st_ref[...]
        Qs = q_ref[:, :O]
        Ql = q_ref[:, O:]
        tot = acc_ref[0]

        def conv_half(mx_g, mn_g, Q, t, t2, gamma, beta, k):
            cnt = pts * k
            mean = t / cnt
            var = t2 / cnt - mean * mean
            a = gamma * lax.rsqrt(var + EPS)
            c = beta - mean * a
            sel = jnp.where(a >= 0, mx_g, mn_g)
            y = a[None, :] * (sel + Q) + c[None, :]
            return jnp.where(y >= 0, y, SLOPE * y)

        ys = conv_half(st[:, 2 * O:3 * O], st[:, 3 * O:4 * O], Qs,
                       tot[0:O], tot[O:2 * O], gs_ref[0], bs_ref[0], K1)
        yl = conv_half(st[:, 6 * O:7 * O], st[:, 7 * O:8 * O], Ql,
                       tot[2 * O:3 * O], tot[3 * O:4 * O],
                       gl_ref[0], bl_ref[0], K2)
        ycat = jnp.concatenate([ys, yl], axis=1)
        yf = lax.dot_general(ycat, wf_ref[...], (((1,), (1,)), ((), ())),
                             preferred_element_type=jnp.float32)
        yf_ref[pl.ds(i * ch, ch), :] = yf
        acc_ref[0, 4 * O:6 * O] = acc_ref[0, 4 * O:6 * O] + jnp.concatenate(
            [jnp.sum(yf, axis=0), jnp.sum(yf * yf, axis=0)])

    @pl.when(p == 2)
    def _fuse():
        tot = acc_ref[0]
        m = tot[4 * O:5 * O] / pts
        v = tot[5 * O:6 * O] / pts - m * m
        a = gf_ref[0] * lax.rsqrt(v + EPS)
        c = bf_ref[0] - m * a
        yf = yf_ref[pl.ds(i * ch, ch), :]
        y = a[None, :] * yf + c[None, :]
        out_ref[...] = jnp.where(y >= 0, y, SLOPE * y)


def kernel(x, W_s, gamma_s, beta_s, W_l, gamma_l, beta_l, W_f, gamma_f, beta_f):
    B, C, N = x.shape
    O = W_s.shape[0]
    R = ROWS
    nR = N // R
    PTS = B * N

    stage1 = pl.pallas_call(
        _stage1_body,
        grid=(nR,),
        in_specs=[
            pl.BlockSpec((1, C, N), lambda r: (0, 0, 0)),
            pl.BlockSpec((1, C, R), lambda r: (0, 0, r)),
            pl.BlockSpec((O, 2 * C), lambda r: (0, 0)),
            pl.BlockSpec((O, 2 * C), lambda r: (0, 0)),
        ],
        out_specs=[
            pl.BlockSpec((1, R, 2 * O), lambda r: (0, r, 0)),
            pl.BlockSpec((1, R, 2 * O), lambda r: (0, r, 0)),
            pl.BlockSpec((1, 1, K2, R), lambda r: (0, r, 0, 0)),
        ],
        out_shape=[
            jax.ShapeDtypeStruct((1, N, 2 * O), jnp.float32),
            jax.ShapeDtypeStruct((1, N, 2 * O), jnp.float32),
            jax.ShapeDtypeStruct((1, nR, K2, R), jnp.int32),
        ],
        scratch_shapes=[pltpu.VMEM((N, R), jnp.float32)],
    )

    stats_parts, q_parts = [], []
    for b in range(B):
        xb = lax.slice_in_dim(x, b, b + 1, axis=0)
        Tb, Qb, idxb = stage1(xb, xb, W_s, W_l)
        idxb_flat = idxb.transpose(0, 1, 3, 2).reshape(-1)
        stats_parts.append(_sc_stage2(Tb.reshape(N, 2 * O), idxb_flat))
        q_parts.append(Qb.reshape(N, 2 * O))
    stats = jnp.concatenate(stats_parts, axis=0)
    Qf = jnp.concatenate(q_parts, axis=0)

    CH = 1024
    nch = PTS // CH
    st_spec = pl.BlockSpec((CH, 8 * O), lambda p, i: (jnp.where(p < 2, i, 0), 0))
    q_spec = pl.BlockSpec((CH, 2 * O), lambda p, i: (jnp.where(p < 2, i, 0), 0))
    full = lambda shape: pl.BlockSpec(shape, lambda p, i: tuple(0 for _ in shape))

    rows = pl.pallas_call(
        functools.partial(_stage3_body, pts=PTS, ch=CH),
        grid=(3, nch),
        in_specs=[st_spec, q_spec, full((O, 2 * O)),
                  full((1, O)), full((1, O)), full((1, O)), full((1, O)),
                  full((1, O)), full((1, O))],
        out_specs=pl.BlockSpec((CH, O), lambda p, i: (i, 0)),
        out_shape=jax.ShapeDtypeStruct((PTS, O), jnp.float32),
        scratch_shapes=[pltpu.VMEM((1, 6 * O), jnp.float32),
                        pltpu.VMEM((PTS, O), jnp.float32)],
    )(stats, Qf, W_f,
      gamma_s.reshape(1, O), beta_s.reshape(1, O),
      gamma_l.reshape(1, O), beta_l.reshape(1, O),
      gamma_f.reshape(1, O), beta_f.reshape(1, O))

    return rows.reshape(B, N, O).transpose(0, 2, 1)
